# baseline (speedup 1.0000x reference)
import functools

import jax
import jax.numpy as jnp
from jax import lax
from jax.experimental import pallas as pl
from jax.experimental.pallas import tpu as pltpu
from jax.experimental.pallas import tpu_sc as plsc

NNODE = 100000
NEDGE = 3200000

NC = 2
NS = 16
NW = NC * NS

RS = 8192
P = (NNODE + RS - 1) // RS
SPAD = RS + 128
NPAD = NNODE + 16
NOUT = P * RS

ET = NEDGE // NW
CH = 2000
FLUSH = 512
STG = FLUSH + 16
NBLK_SEG = ET // FLUSH + 1
SEG = NBLK_SEG * FLUSH
TOTE = NW * P * SEG
NBLKTOT = NW * P * NBLK_SEG
SZCH = 344
WL = 544
DUMMY_BLK = NBLKTOT

_mesh = plsc.VectorSubcoreMesh(core_axis_name="c", subcore_axis_name="s")


def _prefix16(x, iota):
    y = x
    for d in (1, 2, 4, 8):
        idx = jnp.maximum(iota - d, 0)
        sh = y.at[idx].get(mode="promise_in_bounds")
        y = y + jnp.where(iota >= d, sh, 0)
    return y


def _build_worklist(pv, s, cntf, worklist, iota):
    cr0 = cntf[pl.ds(pv * NW, 16)]
    cr1 = cntf[pl.ds(pv * NW + 16, 16)]
    wcnt = jnp.int32(0)
    gbase = jnp.int32(0)
    for t2 in range(NW):
        v = cr0 if t2 < 16 else cr1
        nb = v[t2 % 16]
        b0 = lax.rem(s - gbase, jnp.int32(16))
        b0 = jnp.where(b0 < 0, b0 + 16, b0)
        nmy = jnp.maximum((nb - b0 + 15) // 16, 0)
        cand = (t2 * P) * NBLK_SEG + pv * NBLK_SEG + b0 + iota * 16
        plsc.store_scatter(worklist, [wcnt + iota], cand, mask=iota < nmy)
        wcnt = wcnt + nmy
        gbase = gbase + nb
    npad = lax.rem(jnp.int32(16) - lax.rem(wcnt, jnp.int32(16)), jnp.int32(16))
    plsc.store_scatter(worklist, [wcnt + iota],
                       jnp.full((16,), DUMMY_BLK, jnp.int32),
                       mask=iota < npad)
    return (wcnt + npad) // 16


def _pp_count():
    return (P + NC - 1) // NC


def _bin_body(rows_hbm, cols_hbm, brow_hbm, bcol_hbm, counts_hbm,
              rowch, colch, stg_r, stg_c, cntbuf, scnt):
    c = lax.axis_index("c")
    s = lax.axis_index("s")
    t = c * NS + s
    e0 = t * ET
    iota = lax.iota(jnp.int32, 16)
    drow = jnp.int32(NNODE) + iota
    dcol = jnp.int32(RS) + iota

    for p in range(P):
        scnt[p] = jnp.int32(0)
        scnt[16 + p] = jnp.int32(0)

    def chunk_body(k, carry):
        pltpu.sync_copy(rows_hbm.at[pl.ds(e0 + k * CH, CH)], rowch)
        pltpu.sync_copy(cols_hbm.at[pl.ds(e0 + k * CH, CH)], colch)

        def vec_body(v, carry2):
            r16 = rowch[pl.ds(v * 16, 16)]
            c16 = colch[pl.ds(v * 16, 16)]
            p16 = lax.shift_right_logical(c16, 13)
            l16 = lax.bitwise_and(c16, RS - 1)
            for p in range(P):
                msk = p16 == p
                cnt = scnt[p]
                pc = _prefix16(jnp.where(msk, jnp.int32(1), jnp.int32(0)), iota)
                dest = cnt + pc - 1
                plsc.store_scatter(stg_r[p], [dest], r16, mask=msk)
                plsc.store_scatter(stg_c[p], [dest], l16, mask=msk)
                npop = pc[15]
                cnt2 = cnt + npop
                do_flush = cnt2 >= FLUSH

                @pl.when(do_flush)
                def _():
                    nb = scnt[16 + p]
                    base = (t * P + p) * SEG + nb * FLUSH
                    pltpu.sync_copy(stg_r[p].at[pl.ds(0, FLUSH)],
                                    brow_hbm.at[pl.ds(base, FLUSH)])
                    pltpu.sync_copy(stg_c[p].at[pl.ds(0, FLUSH)],
                                    bcol_hbm.at[pl.ds(base, FLUSH)])
                    rem = cnt2 - FLUSH
                    pm = iota < rem
                    tr = stg_r[p][pl.ds(FLUSH, 16)]
                    tcv = stg_c[p][pl.ds(FLUSH, 16)]
                    plsc.store_scatter(stg_r[p], [iota], tr, mask=pm)
                    plsc.store_scatter(stg_c[p], [iota], tcv, mask=pm)
                    scnt[16 + p] = nb + 1

                scnt[p] = jnp.where(do_flush, cnt2 - FLUSH, cnt2)
            return carry2

        lax.fori_loop(0, CH // 16, vec_body, 0)
        return carry

    lax.fori_loop(0, ET // CH, chunk_body, 0)

    cvec = jnp.zeros((16,), jnp.int32)
    for p in range(P):
        cnt = scnt[p]

        def fill_body(j, carry):
            idx16 = j * 16 + iota
            m = idx16 >= cnt
            cur_r = stg_r[p][pl.ds(j * 16, 16)]
            cur_c = stg_c[p][pl.ds(j * 16, 16)]
            stg_r[p][pl.ds(j * 16, 16)] = jnp.where(m, drow, cur_r)
            stg_c[p][pl.ds(j * 16, 16)] = jnp.where(m, dcol, cur_c)
            return carry

        lax.fori_loop(0, FLUSH // 16, fill_body, 0)
        nb = scnt[16 + p]

        @pl.when(cnt > 0)
        def _():
            base = (t * P + p) * SEG + nb * FLUSH
            pltpu.sync_copy(stg_r[p].at[pl.ds(0, FLUSH)],
                            brow_hbm.at[pl.ds(base, FLUSH)])
            pltpu.sync_copy(stg_c[p].at[pl.ds(0, FLUSH)],
                            bcol_hbm.at[pl.ds(base, FLUSH)])

        nbf = jnp.where(cnt > 0, nb + 1, nb)
        cvec = jnp.where(iota == p, nbf, cvec)

    cntbuf[...] = cvec
    pltpu.sync_copy(cntbuf, counts_hbm.at[t])

    @pl.when(t == 0)
    def _():
        def fillall(j, carry):
            stg_r[0][pl.ds(j * 16, 16)] = drow
            stg_c[0][pl.ds(j * 16, 16)] = dcol
            return carry

        lax.fori_loop(0, FLUSH // 16, fillall, 0)
        pltpu.sync_copy(stg_r[0].at[pl.ds(0, FLUSH)],
                        brow_hbm.at[pl.ds(NW * P * SEG, FLUSH)])
        pltpu.sync_copy(stg_c[0].at[pl.ds(0, FLUSH)],
                        bcol_hbm.at[pl.ds(NW * P * SEG, FLUSH)])


def _bin_edges(rows, cols):
    k = pl.kernel(
        _bin_body,
        out_type=[
            jax.ShapeDtypeStruct((TOTE + FLUSH,), jnp.int32),
            jax.ShapeDtypeStruct((TOTE + FLUSH,), jnp.int32),
            jax.ShapeDtypeStruct((NW, 16), jnp.int32),
        ],
        mesh=_mesh,
        compiler_params=pltpu.CompilerParams(needs_layout_passes=False, use_tc_tiling_on_sc=False),
        scratch_types=[
            pltpu.VMEM((CH,), jnp.int32),
            pltpu.VMEM((CH,), jnp.int32),
            [pltpu.VMEM((STG,), jnp.int32) for _ in range(P)],
            [pltpu.VMEM((STG,), jnp.int32) for _ in range(P)],
            pltpu.VMEM((16,), jnp.int32),
            pltpu.SMEM((32,), jnp.int32),
        ],
    )
    return k(rows, cols)


def _deg_body(bcol_hbm, counts_hbm, deg_hbm,
              colb, ones, zbuf, cntf, worklist, deg_sp, sema):
    c = lax.axis_index("c")
    s = lax.axis_index("s")
    iota = lax.iota(jnp.int32, 16)
    pltpu.sync_copy(counts_hbm, cntf)

    def zb(i, carry):
        zbuf[pl.ds(i * 16, 16)] = jnp.zeros((16,), jnp.float32)
        return carry

    lax.fori_loop(0, 640 // 16, zb, 0)

    def ob(i, carry):
        ones[pl.ds(i * 16, 16)] = jnp.ones((16,), jnp.float32)
        return carry

    lax.fori_loop(0, 128 // 16, ob, 0)

    def pp_body(pp, carry_pp):
        pv = pp * NC + c

        @pl.when(pv < P)
        def _():
            @pl.when(s < SPAD // 640)
            def _():
                pltpu.sync_copy(zbuf, deg_sp.at[pl.ds(s * 640, 640)])
            ngrp = _build_worklist(pv, s, cntf, worklist, iota)
            plsc.subcore_barrier()

            def grp_body(g, carry):
                wv = worklist[pl.ds(g * 16, 16)]
                for j in range(16):
                    blk = wv[j]
                    pltpu.sync_copy(bcol_hbm.at[blk], colb)
                    ds_ = [pltpu.async_copy(ones, deg_sp.at[colb.at[jj]],
                                            sema, add=True) for jj in range(4)]
                    for d in ds_:
                        d.wait()
                return carry

            lax.fori_loop(0, ngrp, grp_body, 0)
            plsc.subcore_barrier()
            pltpu.sync_copy(deg_sp.at[pl.ds(s * 512, 512)],
                            deg_hbm.at[pl.ds(pv * RS + s * 512, 512)])
            plsc.subcore_barrier()

        return carry_pp

    lax.fori_loop(0, _pp_count(), pp_body, 0)


def _degrees(bcol_blk, counts_flat):
    k = pl.kernel(
        _deg_body,
        out_type=[jax.ShapeDtypeStruct((NOUT,), jnp.float32)],
        mesh=_mesh,
        compiler_params=pltpu.CompilerParams(needs_layout_passes=False, use_tc_tiling_on_sc=False),
        scratch_types=[
            pltpu.VMEM((4, 128), jnp.int32),
            pltpu.VMEM((128,), jnp.float32),
            pltpu.VMEM((640,), jnp.float32),
            pltpu.VMEM((P * NW,), jnp.int32),
            pltpu.VMEM((WL,), jnp.int32),
            pltpu.VMEM_SHARED((SPAD,), jnp.float32),
            pltpu.SemaphoreType.DMA,
        ],
    )
    (deg,) = k(bcol_blk, counts_flat)
    return deg


ZCH = 104


def _acc_body(F, D, y_hbm, brow_hbm, bcol_hbm, counts_hbm, s_hbm,
              idxr, idxc, msg, zbuf, cntf, worklist, s_sp, semi, semg, sems):
    c = lax.axis_index("c")
    s = lax.axis_index("s")
    iota = lax.iota(jnp.int32, 16)
    pltpu.sync_copy(counts_hbm, cntf)

    def zb(r, carry):
        for cc in range(F // 16):
            zbuf[r, pl.ds(cc * 16, 16)] = jnp.zeros((16,), jnp.float32)
        return carry

    lax.fori_loop(0, ZCH, zb, 0)

    def pp_body(pp, carry_pp):
        pv = pp * NC + c

        @pl.when(pv < P)
        def _():
            for q in range(5):
                pltpu.sync_copy(zbuf, s_sp.at[pl.ds((s * 5 + q) * ZCH, ZCH)])
            ngrp = _build_worklist(pv, s, cntf, worklist, iota)
            plsc.subcore_barrier()

            def grp_body(g, carry):
                wv = worklist[pl.ds(g * 16, 16)]
                di = []
                for j in range(16):
                    blk = wv[j]
                    di.append(pltpu.async_copy(brow_hbm.at[blk], idxr.at[j],
                                               semi))
                    di.append(pltpu.async_copy(bcol_hbm.at[blk], idxc.at[j],
                                               semi))
                for d in di:
                    d.wait()

                gd = [None] * 16
                sd = [None] * 16

                def issue_gather(j):
                    slot = j % D
                    gd[j] = [pltpu.async_copy(
                        y_hbm.at[idxr.at[j, jj]],
                        msg.at[slot, pl.ds(jj * 128, 128)], semg[slot])
                        for jj in range(4)]

                def issue_scatter(j):
                    slot = j % D
                    for d in gd[j]:
                        d.wait()
                    sd[j] = [pltpu.async_copy(
                        msg.at[slot, pl.ds(jj * 128, 128)],
                        s_sp.at[idxc.at[j, jj]], sems[slot], add=True)
                        for jj in range(4)]

                for j in range(16):
                    if j >= D:
                        for d in sd[j - D]:
                            d.wait()
                    issue_gather(j)
                    if j >= D - 1:
                        issue_scatter(j - (D - 1))
                for j in range(17 - D, 16):
                    issue_scatter(j)
                for j in range(16 - D, 16):
                    for d in sd[j]:
                        d.wait()
                return carry

            lax.fori_loop(0, ngrp, grp_body, 0)
            plsc.subcore_barrier()
            pltpu.sync_copy(s_sp.at[pl.ds(s * 512, 512)],
                            s_hbm.at[pl.ds(pv * RS + s * 512, 512)])
            plsc.subcore_barrier()

        return carry_pp

    lax.fori_loop(0, _pp_count(), pp_body, 0)


def _accumulate(y, brow_blk, bcol_blk, counts_flat, F):
    D = 2 if F > 32 else 3
    k = pl.kernel(
        functools.partial(_acc_body, F, D),
        out_type=[jax.ShapeDtypeStruct((NOUT, F), jnp.float32)],
        mesh=_mesh,
        compiler_params=pltpu.CompilerParams(needs_layout_passes=False, use_tc_tiling_on_sc=False),
        scratch_types=[
            pltpu.VMEM((16, 4, 128), jnp.int32),
            pltpu.VMEM((16, 4, 128), jnp.int32),
            pltpu.VMEM((D, 512, F), jnp.float32),
            pltpu.VMEM((ZCH, F), jnp.float32),
            pltpu.VMEM((P * NW,), jnp.int32),
            pltpu.VMEM((WL,), jnp.int32),
            pltpu.VMEM_SHARED((SPAD, F), jnp.float32),
            pltpu.SemaphoreType.DMA,
            [pltpu.SemaphoreType.DMA for _ in range(D)],
            [pltpu.SemaphoreType.DMA for _ in range(D)],
        ],
    )
    (out,) = k(y, brow_blk, bcol_blk, counts_flat)
    return out


BR = 1024
GRID = (NPAD + BR - 1) // BR


def _tc_first(deg2, x, W1):
    def f(deg_ref, x_ref, w_ref, dis_ref, y_ref):
        dis = lax.rsqrt(deg_ref[...] + 1.0)
        dis_ref[...] = dis
        y_ref[...] = jnp.dot(x_ref[...], w_ref[...],
                             preferred_element_type=jnp.float32) * dis

    return pl.pallas_call(
        f,
        grid=(GRID,),
        in_specs=[
            pl.BlockSpec((BR, 1), lambda i: (i, 0)),
            pl.BlockSpec((BR, 21), lambda i: (i, 0)),
            pl.BlockSpec((21, 32), lambda i: (0, 0)),
        ],
        out_specs=[
            pl.BlockSpec((BR, 1), lambda i: (i, 0)),
            pl.BlockSpec((BR, 32), lambda i: (i, 0)),
        ],
        out_shape=[
            jax.ShapeDtypeStruct((NPAD, 1), jnp.float32),
            jax.ShapeDtypeStruct((NPAD, 32), jnp.float32),
        ],
    )(deg2, x, W1)


def _tc_mid(sarr, y, dis, b, W, Fin, Fout):
    def f(s_ref, y_ref, d_ref, b_ref, w_ref, o_ref):
        d = d_ref[...]
        h = jnp.maximum(d * (s_ref[...] + y_ref[...]) + b_ref[...], 0.0)
        o_ref[...] = jnp.dot(h, w_ref[...],
                             preferred_element_type=jnp.float32) * d

    return pl.pallas_call(
        f,
        grid=(GRID,),
        in_specs=[
            pl.BlockSpec((BR, Fin), lambda i: (i, 0)),
            pl.BlockSpec((BR, Fin), lambda i: (i, 0)),
            pl.BlockSpec((BR, 1), lambda i: (i, 0)),
            pl.BlockSpec((1, Fin), lambda i: (0, 0)),
            pl.BlockSpec((Fin, Fout), lambda i: (0, 0)),
        ],
        out_specs=pl.BlockSpec((BR, Fout), lambda i: (i, 0)),
        out_shape=jax.ShapeDtypeStruct((NPAD, Fout), jnp.float32),
    )(sarr, y, dis, b, W)


def _tc_head(s3, y3, dis, b3, Wl1, bl1, Wl2, bl2):
    def f(s_ref, y_ref, d_ref, b_ref, w1_ref, c1_ref, w2_ref, c2_ref, o_ref):
        d = d_ref[...]
        h = jnp.maximum(d * (s_ref[...] + y_ref[...]) + b_ref[...], 0.0)
        z = jnp.maximum(jnp.dot(h, w1_ref[...],
                                preferred_element_type=jnp.float32)
                        + c1_ref[...], 0.0)
        o_ref[...] = jnp.dot(z, w2_ref[...],
                             preferred_element_type=jnp.float32) + c2_ref[...]

    return pl.pallas_call(
        f,
        grid=(GRID,),
        in_specs=[
            pl.BlockSpec((BR, 32), lambda i: (i, 0)),
            pl.BlockSpec((BR, 32), lambda i: (i, 0)),
            pl.BlockSpec((BR, 1), lambda i: (i, 0)),
            pl.BlockSpec((1, 32), lambda i: (0, 0)),
            pl.BlockSpec((32, 20), lambda i: (0, 0)),
            pl.BlockSpec((1, 20), lambda i: (0, 0)),
            pl.BlockSpec((20, 1), lambda i: (0, 0)),
            pl.BlockSpec((1, 1), lambda i: (0, 0)),
        ],
        out_specs=pl.BlockSpec((BR, 1), lambda i: (i, 0)),
        out_shape=jax.ShapeDtypeStruct((NNODE, 1), jnp.float32),
    )(s3, y3, dis, b3, Wl1, bl1, Wl2, bl2)


def kernel(x, edge_index, W1, b1, W2, b2, W3, b3, Wl1, bl1, Wl2, bl2):
    rows = edge_index[0]
    cols = edge_index[1]

    brow, bcol, counts = _bin_edges(rows, cols)
    brow_blk = brow.reshape(NBLKTOT + 1, 4, 128)
    bcol_blk = bcol.reshape(NBLKTOT + 1, 4, 128)
    counts_flat = counts[:, :P].T.reshape(P * NW)

    deg = _degrees(bcol_blk, counts_flat)

    dis, y1 = _tc_first(deg.reshape(NOUT, 1), x, W1)
    s1 = _accumulate(y1, brow_blk, bcol_blk, counts_flat, 32)
    y2 = _tc_mid(s1, y1, dis, b1.reshape(1, -1), W2, 32, 64)
    s2 = _accumulate(y2, brow_blk, bcol_blk, counts_flat, 64)
    y3 = _tc_mid(s2, y2, dis, b2.reshape(1, -1), W3, 64, 32)
    s3 = _accumulate(y3, brow_blk, bcol_blk, counts_flat, 32)
    return _tc_head(s3, y3, dis, b3.reshape(1, -1), Wl1, bl1.reshape(1, -1),
                    Wl2, bl2.reshape(1, -1))


# speedup vs baseline: 2.9815x; 2.9815x over previous
import functools

import jax
import jax.numpy as jnp
from jax import lax
from jax.experimental import pallas as pl
from jax.experimental.pallas import tpu as pltpu
from jax.experimental.pallas import tpu_sc as plsc

NNODE = 100000
NEDGE = 3200000

NC = 2
NS = 16
NW = NC * NS

RS = 8192
P = (NNODE + RS - 1) // RS
SPAD = RS + 128
NPAD = 100352
NOUT = P * RS

ET = NEDGE // NW
CH = 2000
FLUSH = 512
STG = FLUSH + 16
NBLK_SEG = ET // FLUSH + 1
SEG = NBLK_SEG * FLUSH
TOTE = NW * P * SEG
NBLKTOT = NW * P * NBLK_SEG
SZCH = 344
WL = 544
DUMMY_BLK = NBLKTOT

_mesh = plsc.VectorSubcoreMesh(core_axis_name="c", subcore_axis_name="s")


def _prefix16(x, iota):
    y = x
    for d in (1, 2, 4, 8):
        idx = jnp.maximum(iota - d, 0)
        sh = y.at[idx].get(mode="promise_in_bounds")
        y = y + jnp.where(iota >= d, sh, 0)
    return y


def _build_worklist(pv, s, cntf, worklist, iota):
    cr0 = cntf[pl.ds(pv * NW, 16)]
    cr1 = cntf[pl.ds(pv * NW + 16, 16)]
    wcnt = jnp.int32(0)
    gbase = jnp.int32(0)
    for t2 in range(NW):
        v = cr0 if t2 < 16 else cr1
        nb = v[t2 % 16]
        b0 = lax.rem(s - gbase, jnp.int32(16))
        b0 = jnp.where(b0 < 0, b0 + 16, b0)
        nmy = jnp.maximum((nb - b0 + 15) // 16, 0)
        cand = (t2 * P) * NBLK_SEG + pv * NBLK_SEG + b0 + iota * 16
        plsc.store_scatter(worklist, [wcnt + iota], cand, mask=iota < nmy)
        wcnt = wcnt + nmy
        gbase = gbase + nb
    npad = lax.rem(jnp.int32(16) - lax.rem(wcnt, jnp.int32(16)), jnp.int32(16))
    plsc.store_scatter(worklist, [wcnt + iota],
                       jnp.full((16,), DUMMY_BLK, jnp.int32),
                       mask=iota < npad)
    return (wcnt + npad) // 16


def _pp_count():
    return (P + NC - 1) // NC


def _bin_body(rows_hbm, cols_hbm, brow_hbm, bcol_hbm, counts_hbm,
              rowch, colch, stg_r, stg_c, cntbuf, scnt):
    c = lax.axis_index("c")
    s = lax.axis_index("s")
    t = c * NS + s
    e0 = t * ET
    iota = lax.iota(jnp.int32, 16)
    drow = jnp.int32(NNODE) + iota
    dcol = jnp.int32(RS) + iota

    for p in range(P):
        scnt[p] = jnp.int32(0)
        scnt[16 + p] = jnp.int32(0)

    def chunk_body(k, carry):
        pltpu.sync_copy(rows_hbm.at[pl.ds(e0 + k * CH, CH)], rowch)
        pltpu.sync_copy(cols_hbm.at[pl.ds(e0 + k * CH, CH)], colch)

        def vec_body(v, carry2):
            r16 = rowch[pl.ds(v * 16, 16)]
            c16 = colch[pl.ds(v * 16, 16)]
            p16 = lax.shift_right_logical(c16, 13)
            l16 = lax.bitwise_and(c16, RS - 1)
            for p in range(P):
                msk = p16 == p
                cnt = scnt[p]
                pc = _prefix16(jnp.where(msk, jnp.int32(1), jnp.int32(0)), iota)
                dest = cnt + pc - 1
                plsc.store_scatter(stg_r[p], [dest], r16, mask=msk)
                plsc.store_scatter(stg_c[p], [dest], l16, mask=msk)
                npop = pc[15]
                cnt2 = cnt + npop
                do_flush = cnt2 >= FLUSH

                @pl.when(do_flush)
                def _():
                    nb = scnt[16 + p]
                    base = (t * P + p) * SEG + nb * FLUSH
                    pltpu.sync_copy(stg_r[p].at[pl.ds(0, FLUSH)],
                                    brow_hbm.at[pl.ds(base, FLUSH)])
                    pltpu.sync_copy(stg_c[p].at[pl.ds(0, FLUSH)],
                                    bcol_hbm.at[pl.ds(base, FLUSH)])
                    rem = cnt2 - FLUSH
                    pm = iota < rem
                    tr = stg_r[p][pl.ds(FLUSH, 16)]
                    tcv = stg_c[p][pl.ds(FLUSH, 16)]
                    plsc.store_scatter(stg_r[p], [iota], tr, mask=pm)
                    plsc.store_scatter(stg_c[p], [iota], tcv, mask=pm)
                    scnt[16 + p] = nb + 1

                scnt[p] = jnp.where(do_flush, cnt2 - FLUSH, cnt2)
            return carry2

        lax.fori_loop(0, CH // 16, vec_body, 0)
        return carry

    lax.fori_loop(0, ET // CH, chunk_body, 0)

    cvec = jnp.zeros((16,), jnp.int32)
    for p in range(P):
        cnt = scnt[p]

        def fill_body(j, carry):
            idx16 = j * 16 + iota
            m = idx16 >= cnt
            drow_j = jnp.int32(NNODE) + lax.bitwise_and(idx16 + t * 37,
                                                        jnp.int32(255))
            dcol_j = jnp.int32(RS) + lax.bitwise_and(idx16 + t * 11,
                                                     jnp.int32(127))
            cur_r = stg_r[p][pl.ds(j * 16, 16)]
            cur_c = stg_c[p][pl.ds(j * 16, 16)]
            stg_r[p][pl.ds(j * 16, 16)] = jnp.where(m, drow_j, cur_r)
            stg_c[p][pl.ds(j * 16, 16)] = jnp.where(m, dcol_j, cur_c)
            return carry

        lax.fori_loop(0, FLUSH // 16, fill_body, 0)
        nb = scnt[16 + p]

        @pl.when(cnt > 0)
        def _():
            base = (t * P + p) * SEG + nb * FLUSH
            pltpu.sync_copy(stg_r[p].at[pl.ds(0, FLUSH)],
                            brow_hbm.at[pl.ds(base, FLUSH)])
            pltpu.sync_copy(stg_c[p].at[pl.ds(0, FLUSH)],
                            bcol_hbm.at[pl.ds(base, FLUSH)])

        nbf = jnp.where(cnt > 0, nb + 1, nb)
        cvec = jnp.where(iota == p, nbf, cvec)

    cntbuf[...] = cvec
    pltpu.sync_copy(cntbuf, counts_hbm.at[t])

    @pl.when(t == 0)
    def _():
        def fillall(j, carry):
            idx16 = j * 16 + iota
            stg_r[0][pl.ds(j * 16, 16)] = jnp.int32(NNODE) + lax.bitwise_and(
                idx16, jnp.int32(255))
            stg_c[0][pl.ds(j * 16, 16)] = jnp.int32(RS) + lax.bitwise_and(
                idx16, jnp.int32(127))
            return carry

        lax.fori_loop(0, FLUSH // 16, fillall, 0)
        pltpu.sync_copy(stg_r[0].at[pl.ds(0, FLUSH)],
                        brow_hbm.at[pl.ds(NW * P * SEG, FLUSH)])
        pltpu.sync_copy(stg_c[0].at[pl.ds(0, FLUSH)],
                        bcol_hbm.at[pl.ds(NW * P * SEG, FLUSH)])


def _bin_edges(rows, cols):
    k = pl.kernel(
        _bin_body,
        out_type=[
            jax.ShapeDtypeStruct((TOTE + FLUSH,), jnp.int32),
            jax.ShapeDtypeStruct((TOTE + FLUSH,), jnp.int32),
            jax.ShapeDtypeStruct((NW, 16), jnp.int32),
        ],
        mesh=_mesh,
        compiler_params=pltpu.CompilerParams(needs_layout_passes=False, use_tc_tiling_on_sc=False),
        scratch_types=[
            pltpu.VMEM((CH,), jnp.int32),
            pltpu.VMEM((CH,), jnp.int32),
            [pltpu.VMEM((STG,), jnp.int32) for _ in range(P)],
            [pltpu.VMEM((STG,), jnp.int32) for _ in range(P)],
            pltpu.VMEM((16,), jnp.int32),
            pltpu.SMEM((32,), jnp.int32),
        ],
    )
    return k(rows, cols)


def _deg_body(bcol_hbm, counts_hbm, deg_hbm,
              colb, ones, zbuf, cntf, worklist, deg_sp, sema):
    c = lax.axis_index("c")
    s = lax.axis_index("s")
    iota = lax.iota(jnp.int32, 16)
    pltpu.sync_copy(counts_hbm, cntf)

    def zb(i, carry):
        zbuf[pl.ds(i * 16, 16)] = jnp.zeros((16,), jnp.float32)
        return carry

    lax.fori_loop(0, 640 // 16, zb, 0)

    def ob(i, carry):
        ones[pl.ds(i * 16, 16)] = jnp.ones((16,), jnp.float32)
        return carry

    lax.fori_loop(0, 128 // 16, ob, 0)

    def pp_body(pp, carry_pp):
        pv = pp * NC + c

        @pl.when(pv < P)
        def _():
            @pl.when(s < SPAD // 640)
            def _():
                pltpu.sync_copy(zbuf, deg_sp.at[pl.ds(s * 640, 640)])
            ngrp = _build_worklist(pv, s, cntf, worklist, iota)
            plsc.subcore_barrier()

            def grp_body(g, carry):
                wv = worklist[pl.ds(g * 16, 16)]
                for j in range(16):
                    blk = wv[j]
                    pltpu.sync_copy(bcol_hbm.at[blk], colb)
                    ds_ = [pltpu.async_copy(ones, deg_sp.at[colb.at[jj]],
                                            sema, add=True) for jj in range(4)]
                    for d in ds_:
                        d.wait()
                return carry

            lax.fori_loop(0, ngrp, grp_body, 0)
            plsc.subcore_barrier()
            pltpu.sync_copy(deg_sp.at[pl.ds(s * 512, 512)],
                            deg_hbm.at[pl.ds(pv * RS + s * 512, 512)])
            plsc.subcore_barrier()

        return carry_pp

    lax.fori_loop(0, _pp_count(), pp_body, 0)


def _degrees(bcol_blk, counts_flat):
    k = pl.kernel(
        _deg_body,
        out_type=[jax.ShapeDtypeStruct((NOUT,), jnp.float32)],
        mesh=_mesh,
        compiler_params=pltpu.CompilerParams(needs_layout_passes=False, use_tc_tiling_on_sc=False),
        scratch_types=[
            pltpu.VMEM((4, 128), jnp.int32),
            pltpu.VMEM((128,), jnp.float32),
            pltpu.VMEM((640,), jnp.float32),
            pltpu.VMEM((P * NW,), jnp.int32),
            pltpu.VMEM((WL,), jnp.int32),
            pltpu.VMEM_SHARED((SPAD,), jnp.float32),
            pltpu.SemaphoreType.DMA,
        ],
    )
    (deg,) = k(bcol_blk, counts_flat)
    return deg


ZCH = 104


def _acc_body(F, D, y_hbm, brow_hbm, bcol_hbm, counts_hbm, s_hbm,
              idxr, idxc, msg, zbuf, cntf, worklist, s_sp, semi, semg, sems):
    c = lax.axis_index("c")
    s = lax.axis_index("s")
    iota = lax.iota(jnp.int32, 16)
    pltpu.sync_copy(counts_hbm, cntf)

    def zb(r, carry):
        for cc in range(F // 16):
            zbuf[r, pl.ds(cc * 16, 16)] = jnp.zeros((16,), jnp.float32)
        return carry

    lax.fori_loop(0, ZCH, zb, 0)

    def pp_body(pp, carry_pp):
        pv = pp * NC + c

        @pl.when(pv < P)
        def _():
            for q in range(5):
                pltpu.sync_copy(zbuf, s_sp.at[pl.ds((s * 5 + q) * ZCH, ZCH)])
            ngrp = _build_worklist(pv, s, cntf, worklist, iota)
            plsc.subcore_barrier()

            def grp_body(g, carry):
                wv = worklist[pl.ds(g * 16, 16)]
                di = []
                for j in range(16):
                    blk = wv[j]
                    di.append(pltpu.async_copy(brow_hbm.at[blk], idxr.at[j],
                                               semi))
                    di.append(pltpu.async_copy(bcol_hbm.at[blk], idxc.at[j],
                                               semi))
                for d in di:
                    d.wait()

                gd = [None] * 16
                sd = [None] * 16

                def issue_gather(j):
                    slot = j % D
                    gd[j] = [pltpu.async_copy(
                        y_hbm.at[idxr.at[j, jj]],
                        msg.at[slot, pl.ds(jj * 128, 128)], semg[slot])
                        for jj in range(4)]

                def issue_scatter(j):
                    slot = j % D
                    for d in gd[j]:
                        d.wait()
                    sd[j] = [pltpu.async_copy(
                        msg.at[slot, pl.ds(jj * 128, 128)],
                        s_sp.at[idxc.at[j, jj]], sems[slot], add=True)
                        for jj in range(4)]

                for j in range(16):
                    if j >= D:
                        for d in sd[j - D]:
                            d.wait()
                    issue_gather(j)
                    if j >= D - 1:
                        issue_scatter(j - (D - 1))
                for j in range(17 - D, 16):
                    issue_scatter(j)
                for j in range(16 - D, 16):
                    for d in sd[j]:
                        d.wait()
                return carry

            lax.fori_loop(0, ngrp, grp_body, 0)
            plsc.subcore_barrier()
            pltpu.sync_copy(s_sp.at[pl.ds(s * 512, 512)],
                            s_hbm.at[pl.ds(pv * RS + s * 512, 512)])
            plsc.subcore_barrier()

        return carry_pp

    lax.fori_loop(0, _pp_count(), pp_body, 0)


def _accumulate(y, brow_blk, bcol_blk, counts_flat, F):
    D = 2 if F > 32 else 3
    k = pl.kernel(
        functools.partial(_acc_body, F, D),
        out_type=[jax.ShapeDtypeStruct((NOUT, F), jnp.float32)],
        mesh=_mesh,
        compiler_params=pltpu.CompilerParams(needs_layout_passes=False, use_tc_tiling_on_sc=False),
        scratch_types=[
            pltpu.VMEM((16, 4, 128), jnp.int32),
            pltpu.VMEM((16, 4, 128), jnp.int32),
            pltpu.VMEM((D, 512, F), jnp.float32),
            pltpu.VMEM((ZCH, F), jnp.float32),
            pltpu.VMEM((P * NW,), jnp.int32),
            pltpu.VMEM((WL,), jnp.int32),
            pltpu.VMEM_SHARED((SPAD, F), jnp.float32),
            pltpu.SemaphoreType.DMA,
            [pltpu.SemaphoreType.DMA for _ in range(D)],
            [pltpu.SemaphoreType.DMA for _ in range(D)],
        ],
    )
    (out,) = k(y, brow_blk, bcol_blk, counts_flat)
    return out


BR = 1024
GRID = (NPAD + BR - 1) // BR


def _tc_first(deg2, x, W1):
    def f(deg_ref, x_ref, w_ref, dis_ref, y_ref):
        dis = lax.rsqrt(deg_ref[...] + 1.0)
        dis_ref[...] = dis
        y_ref[...] = jnp.dot(x_ref[...], w_ref[...],
                             preferred_element_type=jnp.float32) * dis

    return pl.pallas_call(
        f,
        grid=(GRID,),
        in_specs=[
            pl.BlockSpec((BR, 1), lambda i: (i, 0)),
            pl.BlockSpec((BR, 21), lambda i: (i, 0)),
            pl.BlockSpec((21, 32), lambda i: (0, 0)),
        ],
        out_specs=[
            pl.BlockSpec((BR, 1), lambda i: (i, 0)),
            pl.BlockSpec((BR, 32), lambda i: (i, 0)),
        ],
        out_shape=[
            jax.ShapeDtypeStruct((NPAD, 1), jnp.float32),
            jax.ShapeDtypeStruct((NPAD, 32), jnp.float32),
        ],
    )(deg2, x, W1)


def _tc_mid(sarr, y, dis, b, W, Fin, Fout):
    def f(s_ref, y_ref, d_ref, b_ref, w_ref, o_ref):
        d = d_ref[...]
        h = jnp.maximum(d * (s_ref[...] + y_ref[...]) + b_ref[...], 0.0)
        o_ref[...] = jnp.dot(h, w_ref[...],
                             preferred_element_type=jnp.float32) * d

    return pl.pallas_call(
        f,
        grid=(GRID,),
        in_specs=[
            pl.BlockSpec((BR, Fin), lambda i: (i, 0)),
            pl.BlockSpec((BR, Fin), lambda i: (i, 0)),
            pl.BlockSpec((BR, 1), lambda i: (i, 0)),
            pl.BlockSpec((1, Fin), lambda i: (0, 0)),
            pl.BlockSpec((Fin, Fout), lambda i: (0, 0)),
        ],
        out_specs=pl.BlockSpec((BR, Fout), lambda i: (i, 0)),
        out_shape=jax.ShapeDtypeStruct((NPAD, Fout), jnp.float32),
    )(sarr, y, dis, b, W)


def _tc_head(s3, y3, dis, b3, Wl1, bl1, Wl2, bl2):
    def f(s_ref, y_ref, d_ref, b_ref, w1_ref, c1_ref, w2_ref, c2_ref, o_ref):
        d = d_ref[...]
        h = jnp.maximum(d * (s_ref[...] + y_ref[...]) + b_ref[...], 0.0)
        z = jnp.maximum(jnp.dot(h, w1_ref[...],
                                preferred_element_type=jnp.float32)
                        + c1_ref[...], 0.0)
        o_ref[...] = jnp.dot(z, w2_ref[...],
                             preferred_element_type=jnp.float32) + c2_ref[...]

    return pl.pallas_call(
        f,
        grid=(GRID,),
        in_specs=[
            pl.BlockSpec((BR, 32), lambda i: (i, 0)),
            pl.BlockSpec((BR, 32), lambda i: (i, 0)),
            pl.BlockSpec((BR, 1), lambda i: (i, 0)),
            pl.BlockSpec((1, 32), lambda i: (0, 0)),
            pl.BlockSpec((32, 20), lambda i: (0, 0)),
            pl.BlockSpec((1, 20), lambda i: (0, 0)),
            pl.BlockSpec((20, 1), lambda i: (0, 0)),
            pl.BlockSpec((1, 1), lambda i: (0, 0)),
        ],
        out_specs=pl.BlockSpec((BR, 1), lambda i: (i, 0)),
        out_shape=jax.ShapeDtypeStruct((NNODE, 1), jnp.float32),
    )(s3, y3, dis, b3, Wl1, bl1, Wl2, bl2)


def kernel(x, edge_index, W1, b1, W2, b2, W3, b3, Wl1, bl1, Wl2, bl2):
    rows = edge_index[0]
    cols = edge_index[1]

    brow, bcol, counts = _bin_edges(rows, cols)
    brow_blk = brow.reshape(NBLKTOT + 1, 4, 128)
    bcol_blk = bcol.reshape(NBLKTOT + 1, 4, 128)
    counts_flat = counts[:, :P].T.reshape(P * NW)

    deg = _degrees(bcol_blk, counts_flat)

    dis, y1 = _tc_first(deg.reshape(NOUT, 1), x, W1)
    s1 = _accumulate(y1, brow_blk, bcol_blk, counts_flat, 32)
    y2 = _tc_mid(s1, y1, dis, b1.reshape(1, -1), W2, 32, 64)
    s2 = _accumulate(y2, brow_blk, bcol_blk, counts_flat, 64)
    y3 = _tc_mid(s2, y2, dis, b2.reshape(1, -1), W3, 64, 32)
    s3 = _accumulate(y3, brow_blk, bcol_blk, counts_flat, 32)
    return _tc_head(s3, y3, dis, b3.reshape(1, -1), Wl1, bl1.reshape(1, -1),
                    Wl2, bl2.reshape(1, -1))


# speedup vs baseline: 3.5131x; 1.1783x over previous
import functools

import jax
import jax.numpy as jnp
from jax import lax
from jax.experimental import pallas as pl
from jax.experimental.pallas import tpu as pltpu
from jax.experimental.pallas import tpu_sc as plsc

NNODE = 100000
NEDGE = 3200000

NC = 2
NS = 16
NW = NC * NS

RS = 8192
P = (NNODE + RS - 1) // RS
SPAD = RS + 128
NPAD = 100352
NOUT = P * RS

ET = NEDGE // NW
CH = 2000
FLUSH = 512
STG = FLUSH + 16
NBLK_SEG = ET // FLUSH + 1
SEG = NBLK_SEG * FLUSH
TOTE = NW * P * SEG
NBLKTOT = NW * P * NBLK_SEG
SZCH = 344
WL = 544
DUMMY_BLK = NBLKTOT

_mesh = plsc.VectorSubcoreMesh(core_axis_name="c", subcore_axis_name="s")


def _prefix16(x, iota):
    y = x
    for d in (1, 2, 4, 8):
        idx = jnp.maximum(iota - d, 0)
        sh = y.at[idx].get(mode="promise_in_bounds")
        y = y + jnp.where(iota >= d, sh, 0)
    return y


def _build_worklist(pv, s, cntf, worklist, iota):
    cr0 = cntf[pl.ds(pv * NW, 16)]
    cr1 = cntf[pl.ds(pv * NW + 16, 16)]
    wcnt = jnp.int32(0)
    gbase = jnp.int32(0)
    for t2 in range(NW):
        v = cr0 if t2 < 16 else cr1
        nb = v[t2 % 16]
        b0 = lax.rem(s - gbase, jnp.int32(16))
        b0 = jnp.where(b0 < 0, b0 + 16, b0)
        nmy = jnp.maximum((nb - b0 + 15) // 16, 0)
        cand = (t2 * P) * NBLK_SEG + pv * NBLK_SEG + b0 + iota * 16
        plsc.store_scatter(worklist, [wcnt + iota], cand, mask=iota < nmy)
        wcnt = wcnt + nmy
        gbase = gbase + nb
    npad = lax.rem(jnp.int32(16) - lax.rem(wcnt, jnp.int32(16)), jnp.int32(16))
    plsc.store_scatter(worklist, [wcnt + iota],
                       jnp.full((16,), DUMMY_BLK, jnp.int32),
                       mask=iota < npad)
    return (wcnt + npad) // 16


def _pp_count():
    return (P + NC - 1) // NC


def _bin_body(rows_hbm, cols_hbm, brow_hbm, bcol_hbm, counts_hbm,
              rowch, colch, stgf_r, stgf_c, cntbuf, scnt):
    c = lax.axis_index("c")
    s = lax.axis_index("s")
    t = c * NS + s
    e0 = t * ET
    iota = lax.iota(jnp.int32, 16)
    ones16 = jnp.full((16,), 1, jnp.int32)

    for p in range(P):
        scnt[16 + p] = jnp.int32(0)

    def chunk_body(k, cnt_vec):
        pltpu.sync_copy(rows_hbm.at[pl.ds(e0 + k * CH, CH)], rowch)
        pltpu.sync_copy(cols_hbm.at[pl.ds(e0 + k * CH, CH)], colch)

        def vec_body(v, cv):
            r16 = rowch[pl.ds(v * 16, 16)]
            c16 = colch[pl.ds(v * 16, 16)]
            p16 = lax.shift_right_logical(c16, 13)
            l16 = lax.bitwise_and(c16, RS - 1)
            isA = p16 <= 5
            isB = (p16 >= 6) & (p16 <= 11)
            shA = jnp.where(isA, 5 * p16, 0)
            shB = jnp.where(isB, 5 * (p16 - 6), 0)
            encA = jnp.where(isA, lax.shift_left(ones16, shA), 0)
            encB = jnp.where(isB, lax.shift_left(ones16, shB), 0)
            encC = jnp.where(p16 == 12, ones16, 0)
            pA = _prefix16(encA, iota)
            pB = _prefix16(encB, iota)
            pC = _prefix16(encC, iota)
            rank = jnp.where(
                isA, lax.bitwise_and(lax.shift_right_logical(pA, shA), 31),
                jnp.where(
                    isB, lax.bitwise_and(lax.shift_right_logical(pB, shB), 31),
                    pC))
            cntg = cv.at[p16].get(mode="promise_in_bounds")
            dest = p16 * STG + cntg + rank - 1
            plsc.store_scatter(stgf_r, [dest], r16)
            plsc.store_scatter(stgf_c, [dest], l16)
            lastA = pA[15]
            lastB = pB[15]
            lastC = pC[15]
            sh1 = 5 * jnp.minimum(iota, 5)
            sh2 = 5 * jnp.clip(iota - 6, 0, 5)
            hist = jnp.where(
                iota <= 5,
                lax.bitwise_and(lax.shift_right_logical(lastA, sh1), 31),
                jnp.where(
                    iota <= 11,
                    lax.bitwise_and(lax.shift_right_logical(lastB, sh2), 31),
                    jnp.where(iota == 12, lastC, 0)))
            cv2 = cv + hist
            mx = cv2
            for d in (8, 4, 2, 1):
                mx = jnp.maximum(
                    mx, mx.at[lax.bitwise_xor(iota, jnp.int32(d))].get(
                        mode="promise_in_bounds"))
            anyf = mx[0] >= FLUSH

            @pl.when(anyf)
            def _():
                for p in range(P):
                    cntp = cv2[p]

                    @pl.when(cntp >= FLUSH)
                    def _():
                        nb = scnt[16 + p]
                        base = (t * P + p) * SEG + nb * FLUSH
                        pltpu.sync_copy(stgf_r.at[pl.ds(p * STG, FLUSH)],
                                        brow_hbm.at[pl.ds(base, FLUSH)])
                        pltpu.sync_copy(stgf_c.at[pl.ds(p * STG, FLUSH)],
                                        bcol_hbm.at[pl.ds(base, FLUSH)])
                        rem = cntp - FLUSH
                        pm = iota < rem
                        tr = stgf_r[pl.ds(p * STG + FLUSH, 16)]
                        tcv = stgf_c[pl.ds(p * STG + FLUSH, 16)]
                        plsc.store_scatter(stgf_r, [p * STG + iota], tr,
                                           mask=pm)
                        plsc.store_scatter(stgf_c, [p * STG + iota], tcv,
                                           mask=pm)
                        scnt[16 + p] = nb + 1

            return jnp.where(cv2 >= FLUSH, cv2 - FLUSH, cv2)

        return lax.fori_loop(0, CH // 16, vec_body, cnt_vec)

    cnt_vec = lax.fori_loop(0, ET // CH, chunk_body,
                            jnp.zeros((16,), jnp.int32))

    cvec = jnp.zeros((16,), jnp.int32)
    for p in range(P):
        cnt = cnt_vec[p]

        def fill_body(j, carry):
            idx16 = j * 16 + iota
            m = idx16 >= cnt
            drow_j = jnp.int32(NNODE) + lax.bitwise_and(idx16 + t * 37,
                                                        jnp.int32(255))
            dcol_j = jnp.int32(RS) + lax.bitwise_and(idx16 + t * 11,
                                                     jnp.int32(127))
            cur_r = stgf_r[pl.ds(p * STG + j * 16, 16)]
            cur_c = stgf_c[pl.ds(p * STG + j * 16, 16)]
            stgf_r[pl.ds(p * STG + j * 16, 16)] = jnp.where(m, drow_j, cur_r)
            stgf_c[pl.ds(p * STG + j * 16, 16)] = jnp.where(m, dcol_j, cur_c)
            return carry

        lax.fori_loop(0, FLUSH // 16, fill_body, 0)
        nb = scnt[16 + p]

        @pl.when(cnt > 0)
        def _():
            base = (t * P + p) * SEG + nb * FLUSH
            pltpu.sync_copy(stgf_r.at[pl.ds(p * STG, FLUSH)],
                            brow_hbm.at[pl.ds(base, FLUSH)])
            pltpu.sync_copy(stgf_c.at[pl.ds(p * STG, FLUSH)],
                            bcol_hbm.at[pl.ds(base, FLUSH)])

        nbf = jnp.where(cnt > 0, nb + 1, nb)
        cvec = jnp.where(iota == p, nbf, cvec)

    cntbuf[...] = cvec
    pltpu.sync_copy(cntbuf, counts_hbm.at[t])

    @pl.when(t == 0)
    def _():
        def fillall(j, carry):
            idx16 = j * 16 + iota
            stgf_r[pl.ds(j * 16, 16)] = jnp.int32(NNODE) + lax.bitwise_and(
                idx16, jnp.int32(255))
            stgf_c[pl.ds(j * 16, 16)] = jnp.int32(RS) + lax.bitwise_and(
                idx16, jnp.int32(127))
            return carry

        lax.fori_loop(0, FLUSH // 16, fillall, 0)
        pltpu.sync_copy(stgf_r.at[pl.ds(0, FLUSH)],
                        brow_hbm.at[pl.ds(NW * P * SEG, FLUSH)])
        pltpu.sync_copy(stgf_c.at[pl.ds(0, FLUSH)],
                        bcol_hbm.at[pl.ds(NW * P * SEG, FLUSH)])


def _bin_edges(rows, cols):
    k = pl.kernel(
        _bin_body,
        out_type=[
            jax.ShapeDtypeStruct((TOTE + FLUSH,), jnp.int32),
            jax.ShapeDtypeStruct((TOTE + FLUSH,), jnp.int32),
            jax.ShapeDtypeStruct((NW, 16), jnp.int32),
        ],
        mesh=_mesh,
        compiler_params=pltpu.CompilerParams(needs_layout_passes=False, use_tc_tiling_on_sc=False),
        scratch_types=[
            pltpu.VMEM((CH,), jnp.int32),
            pltpu.VMEM((CH,), jnp.int32),
            pltpu.VMEM((P * STG,), jnp.int32),
            pltpu.VMEM((P * STG,), jnp.int32),
            pltpu.VMEM((16,), jnp.int32),
            pltpu.SMEM((32,), jnp.int32),
        ],
    )
    return k(rows, cols)


def _deg_body(bcol_hbm, counts_hbm, deg_hbm,
              colb, ones, zbuf, cntf, worklist, deg_sp, sema):
    c = lax.axis_index("c")
    s = lax.axis_index("s")
    iota = lax.iota(jnp.int32, 16)
    pltpu.sync_copy(counts_hbm, cntf)

    def zb(i, carry):
        zbuf[pl.ds(i * 16, 16)] = jnp.zeros((16,), jnp.float32)
        return carry

    lax.fori_loop(0, 640 // 16, zb, 0)

    def ob(i, carry):
        ones[pl.ds(i * 16, 16)] = jnp.ones((16,), jnp.float32)
        return carry

    lax.fori_loop(0, 128 // 16, ob, 0)

    def pp_body(pp, carry_pp):
        pv = pp * NC + c

        @pl.when(pv < P)
        def _():
            @pl.when(s < SPAD // 640)
            def _():
                pltpu.sync_copy(zbuf, deg_sp.at[pl.ds(s * 640, 640)])
            ngrp = _build_worklist(pv, s, cntf, worklist, iota)
            plsc.subcore_barrier()

            def grp_body(g, carry):
                wv = worklist[pl.ds(g * 16, 16)]
                for j in range(16):
                    blk = wv[j]
                    pltpu.sync_copy(bcol_hbm.at[blk], colb)
                    ds_ = [pltpu.async_copy(ones, deg_sp.at[colb.at[jj]],
                                            sema, add=True) for jj in range(4)]
                    for d in ds_:
                        d.wait()
                return carry

            lax.fori_loop(0, ngrp, grp_body, 0)
            plsc.subcore_barrier()
            pltpu.sync_copy(deg_sp.at[pl.ds(s * 512, 512)],
                            deg_hbm.at[pl.ds(pv * RS + s * 512, 512)])
            plsc.subcore_barrier()

        return carry_pp

    lax.fori_loop(0, _pp_count(), pp_body, 0)


def _degrees(bcol_blk, counts_flat):
    k = pl.kernel(
        _deg_body,
        out_type=[jax.ShapeDtypeStruct((NOUT,), jnp.float32)],
        mesh=_mesh,
        compiler_params=pltpu.CompilerParams(needs_layout_passes=False, use_tc_tiling_on_sc=False),
        scratch_types=[
            pltpu.VMEM((4, 128), jnp.int32),
            pltpu.VMEM((128,), jnp.float32),
            pltpu.VMEM((640,), jnp.float32),
            pltpu.VMEM((P * NW,), jnp.int32),
            pltpu.VMEM((WL,), jnp.int32),
            pltpu.VMEM_SHARED((SPAD,), jnp.float32),
            pltpu.SemaphoreType.DMA,
        ],
    )
    (deg,) = k(bcol_blk, counts_flat)
    return deg


ZCH = 104


def _acc_body(F, D, y_hbm, brow_hbm, bcol_hbm, counts_hbm, s_hbm,
              idxr, idxc, msg, zbuf, cntf, worklist, s_sp, semi, semg, sems):
    c = lax.axis_index("c")
    s = lax.axis_index("s")
    iota = lax.iota(jnp.int32, 16)
    pltpu.sync_copy(counts_hbm, cntf)

    def zb(r, carry):
        for cc in range(F // 16):
            zbuf[r, pl.ds(cc * 16, 16)] = jnp.zeros((16,), jnp.float32)
        return carry

    lax.fori_loop(0, ZCH, zb, 0)

    def pp_body(pp, carry_pp):
        pv = pp * NC + c

        @pl.when(pv < P)
        def _():
            for q in range(5):
                pltpu.sync_copy(zbuf, s_sp.at[pl.ds((s * 5 + q) * ZCH, ZCH)])
            ngrp = _build_worklist(pv, s, cntf, worklist, iota)
            plsc.subcore_barrier()

            def grp_body(g, carry):
                wv = worklist[pl.ds(g * 16, 16)]
                di = []
                for j in range(16):
                    blk = wv[j]
                    di.append(pltpu.async_copy(brow_hbm.at[blk], idxr.at[j],
                                               semi))
                    di.append(pltpu.async_copy(bcol_hbm.at[blk], idxc.at[j],
                                               semi))
                for d in di:
                    d.wait()

                gd = [None] * 16
                sd = [None] * 16

                def issue_gather(j):
                    slot = j % D
                    gd[j] = [pltpu.async_copy(
                        y_hbm.at[idxr.at[j, jj]],
                        msg.at[slot, pl.ds(jj * 128, 128)], semg[slot])
                        for jj in range(4)]

                def issue_scatter(j):
                    slot = j % D
                    for d in gd[j]:
                        d.wait()
                    sd[j] = [pltpu.async_copy(
                        msg.at[slot, pl.ds(jj * 128, 128)],
                        s_sp.at[idxc.at[j, jj]], sems[slot], add=True)
                        for jj in range(4)]

                for j in range(16):
                    if j >= D:
                        for d in sd[j - D]:
                            d.wait()
                    issue_gather(j)
                    if j >= D - 1:
                        issue_scatter(j - (D - 1))
                for j in range(17 - D, 16):
                    issue_scatter(j)
                for j in range(16 - D, 16):
                    for d in sd[j]:
                        d.wait()
                return carry

            lax.fori_loop(0, ngrp, grp_body, 0)
            plsc.subcore_barrier()
            pltpu.sync_copy(s_sp.at[pl.ds(s * 512, 512)],
                            s_hbm.at[pl.ds(pv * RS + s * 512, 512)])
            plsc.subcore_barrier()

        return carry_pp

    lax.fori_loop(0, _pp_count(), pp_body, 0)


def _accumulate(y, brow_blk, bcol_blk, counts_flat, F):
    D = 2 if F > 32 else 3
    k = pl.kernel(
        functools.partial(_acc_body, F, D),
        out_type=[jax.ShapeDtypeStruct((NOUT, F), jnp.float32)],
        mesh=_mesh,
        compiler_params=pltpu.CompilerParams(needs_layout_passes=False, use_tc_tiling_on_sc=False),
        scratch_types=[
            pltpu.VMEM((16, 4, 128), jnp.int32),
            pltpu.VMEM((16, 4, 128), jnp.int32),
            pltpu.VMEM((D, 512, F), jnp.float32),
            pltpu.VMEM((ZCH, F), jnp.float32),
            pltpu.VMEM((P * NW,), jnp.int32),
            pltpu.VMEM((WL,), jnp.int32),
            pltpu.VMEM_SHARED((SPAD, F), jnp.float32),
            pltpu.SemaphoreType.DMA,
            [pltpu.SemaphoreType.DMA for _ in range(D)],
            [pltpu.SemaphoreType.DMA for _ in range(D)],
        ],
    )
    (out,) = k(y, brow_blk, bcol_blk, counts_flat)
    return out


BR = 1024
GRID = (NPAD + BR - 1) // BR


def _tc_first(deg2, x, W1):
    def f(deg_ref, x_ref, w_ref, dis_ref, y_ref):
        dis = lax.rsqrt(deg_ref[...] + 1.0)
        dis_ref[...] = dis
        y_ref[...] = jnp.dot(x_ref[...], w_ref[...],
                             preferred_element_type=jnp.float32) * dis

    return pl.pallas_call(
        f,
        grid=(GRID,),
        in_specs=[
            pl.BlockSpec((BR, 1), lambda i: (i, 0)),
            pl.BlockSpec((BR, 21), lambda i: (i, 0)),
            pl.BlockSpec((21, 32), lambda i: (0, 0)),
        ],
        out_specs=[
            pl.BlockSpec((BR, 1), lambda i: (i, 0)),
            pl.BlockSpec((BR, 32), lambda i: (i, 0)),
        ],
        out_shape=[
            jax.ShapeDtypeStruct((NPAD, 1), jnp.float32),
            jax.ShapeDtypeStruct((NPAD, 32), jnp.float32),
        ],
    )(deg2, x, W1)


def _tc_mid(sarr, y, dis, b, W, Fin, Fout):
    def f(s_ref, y_ref, d_ref, b_ref, w_ref, o_ref):
        d = d_ref[...]
        h = jnp.maximum(d * (s_ref[...] + y_ref[...]) + b_ref[...], 0.0)
        o_ref[...] = jnp.dot(h, w_ref[...],
                             preferred_element_type=jnp.float32) * d

    return pl.pallas_call(
        f,
        grid=(GRID,),
        in_specs=[
            pl.BlockSpec((BR, Fin), lambda i: (i, 0)),
            pl.BlockSpec((BR, Fin), lambda i: (i, 0)),
            pl.BlockSpec((BR, 1), lambda i: (i, 0)),
            pl.BlockSpec((1, Fin), lambda i: (0, 0)),
            pl.BlockSpec((Fin, Fout), lambda i: (0, 0)),
        ],
        out_specs=pl.BlockSpec((BR, Fout), lambda i: (i, 0)),
        out_shape=jax.ShapeDtypeStruct((NPAD, Fout), jnp.float32),
    )(sarr, y, dis, b, W)


def _tc_head(s3, y3, dis, b3, Wl1, bl1, Wl2, bl2):
    def f(s_ref, y_ref, d_ref, b_ref, w1_ref, c1_ref, w2_ref, c2_ref, o_ref):
        d = d_ref[...]
        h = jnp.maximum(d * (s_ref[...] + y_ref[...]) + b_ref[...], 0.0)
        z = jnp.maximum(jnp.dot(h, w1_ref[...],
                                preferred_element_type=jnp.float32)
                        + c1_ref[...], 0.0)
        o_ref[...] = jnp.dot(z, w2_ref[...],
                             preferred_element_type=jnp.float32) + c2_ref[...]

    return pl.pallas_call(
        f,
        grid=(GRID,),
        in_specs=[
            pl.BlockSpec((BR, 32), lambda i: (i, 0)),
            pl.BlockSpec((BR, 32), lambda i: (i, 0)),
            pl.BlockSpec((BR, 1), lambda i: (i, 0)),
            pl.BlockSpec((1, 32), lambda i: (0, 0)),
            pl.BlockSpec((32, 20), lambda i: (0, 0)),
            pl.BlockSpec((1, 20), lambda i: (0, 0)),
            pl.BlockSpec((20, 1), lambda i: (0, 0)),
            pl.BlockSpec((1, 1), lambda i: (0, 0)),
        ],
        out_specs=pl.BlockSpec((BR, 1), lambda i: (i, 0)),
        out_shape=jax.ShapeDtypeStruct((NNODE, 1), jnp.float32),
    )(s3, y3, dis, b3, Wl1, bl1, Wl2, bl2)


def kernel(x, edge_index, W1, b1, W2, b2, W3, b3, Wl1, bl1, Wl2, bl2):
    rows = edge_index[0]
    cols = edge_index[1]

    brow, bcol, counts = _bin_edges(rows, cols)
    brow_blk = brow.reshape(NBLKTOT + 1, 4, 128)
    bcol_blk = bcol.reshape(NBLKTOT + 1, 4, 128)
    counts_flat = counts[:, :P].T.reshape(P * NW)

    deg = _degrees(bcol_blk, counts_flat)

    dis, y1 = _tc_first(deg.reshape(NOUT, 1), x, W1)
    s1 = _accumulate(y1, brow_blk, bcol_blk, counts_flat, 32)
    y2 = _tc_mid(s1, y1, dis, b1.reshape(1, -1), W2, 32, 64)
    s2 = _accumulate(y2, brow_blk, bcol_blk, counts_flat, 64)
    y3 = _tc_mid(s2, y2, dis, b2.reshape(1, -1), W3, 64, 32)
    s3 = _accumulate(y3, brow_blk, bcol_blk, counts_flat, 32)
    return _tc_head(s3, y3, dis, b3.reshape(1, -1), Wl1, bl1.reshape(1, -1),
                    Wl2, bl2.reshape(1, -1))


# speedup vs baseline: 3.5180x; 1.0014x over previous
import functools

import jax
import jax.numpy as jnp
from jax import lax
from jax.experimental import pallas as pl
from jax.experimental.pallas import tpu as pltpu
from jax.experimental.pallas import tpu_sc as plsc

NNODE = 100000
NEDGE = 3200000

NC = 2
NS = 16
NW = NC * NS

RS = 8192
P = (NNODE + RS - 1) // RS
SPAD = RS + 128
NPAD = 100352
NOUT = P * RS

ET = NEDGE // NW
CH = 2000
FLUSH = 512
STG = FLUSH + 16
NBLK_SEG = ET // FLUSH + 1
SEG = NBLK_SEG * FLUSH
TOTE = NW * P * SEG
NBLKTOT = NW * P * NBLK_SEG
SZCH = 344
WL = 544
DUMMY_BLK = NBLKTOT

_mesh = plsc.VectorSubcoreMesh(core_axis_name="c", subcore_axis_name="s")


def _prefix16(x, iota):
    y = x
    for d in (1, 2, 4, 8):
        idx = jnp.maximum(iota - d, 0)
        sh = y.at[idx].get(mode="promise_in_bounds")
        y = y + jnp.where(iota >= d, sh, 0)
    return y


def _build_worklist(pv, s, cntf, worklist, iota):
    cr0 = cntf[pl.ds(pv * NW, 16)]
    cr1 = cntf[pl.ds(pv * NW + 16, 16)]
    wcnt = jnp.int32(0)
    gbase = jnp.int32(0)
    for t2 in range(NW):
        v = cr0 if t2 < 16 else cr1
        nb = v[t2 % 16]
        b0 = lax.rem(s - gbase, jnp.int32(16))
        b0 = jnp.where(b0 < 0, b0 + 16, b0)
        nmy = jnp.maximum((nb - b0 + 15) // 16, 0)
        cand = (t2 * P) * NBLK_SEG + pv * NBLK_SEG + b0 + iota * 16
        plsc.store_scatter(worklist, [wcnt + iota], cand, mask=iota < nmy)
        wcnt = wcnt + nmy
        gbase = gbase + nb
    npad = lax.rem(jnp.int32(16) - lax.rem(wcnt, jnp.int32(16)), jnp.int32(16))
    plsc.store_scatter(worklist, [wcnt + iota],
                       jnp.full((16,), DUMMY_BLK, jnp.int32),
                       mask=iota < npad)
    return (wcnt + npad) // 16


def _pp_count():
    return (P + NC - 1) // NC


def _bin_body(rows_hbm, cols_hbm, brow_hbm, bcol_hbm, counts_hbm,
              rowch, colch, stgf_r, stgf_c, cntbuf, scnt):
    c = lax.axis_index("c")
    s = lax.axis_index("s")
    t = c * NS + s
    e0 = t * ET
    iota = lax.iota(jnp.int32, 16)
    ones16 = jnp.full((16,), 1, jnp.int32)

    for p in range(P):
        scnt[16 + p] = jnp.int32(0)

    def chunk_body(k, cnt_vec):
        pltpu.sync_copy(rows_hbm.at[pl.ds(e0 + k * CH, CH)], rowch)
        pltpu.sync_copy(cols_hbm.at[pl.ds(e0 + k * CH, CH)], colch)

        def vec_body(v, cv):
            r16 = rowch[pl.ds(v * 16, 16)]
            c16 = colch[pl.ds(v * 16, 16)]
            p16 = lax.shift_right_logical(c16, 13)
            l16 = lax.bitwise_and(c16, RS - 1)
            isA = p16 <= 5
            isB = (p16 >= 6) & (p16 <= 11)
            shA = jnp.where(isA, 5 * p16, 0)
            shB = jnp.where(isB, 5 * (p16 - 6), 0)
            encA = jnp.where(isA, lax.shift_left(ones16, shA), 0)
            encB = jnp.where(isB, lax.shift_left(ones16, shB), 0)
            encC = jnp.where(p16 == 12, ones16, 0)
            pA = _prefix16(encA, iota)
            pB = _prefix16(encB, iota)
            pC = _prefix16(encC, iota)
            rank = jnp.where(
                isA, lax.bitwise_and(lax.shift_right_logical(pA, shA), 31),
                jnp.where(
                    isB, lax.bitwise_and(lax.shift_right_logical(pB, shB), 31),
                    pC))
            cntg = cv.at[p16].get(mode="promise_in_bounds")
            dest = p16 * STG + cntg + rank - 1
            plsc.store_scatter(stgf_r, [dest], r16)
            plsc.store_scatter(stgf_c, [dest], l16)
            lastA = pA[15]
            lastB = pB[15]
            lastC = pC[15]
            sh1 = 5 * jnp.minimum(iota, 5)
            sh2 = 5 * jnp.clip(iota - 6, 0, 5)
            hist = jnp.where(
                iota <= 5,
                lax.bitwise_and(lax.shift_right_logical(lastA, sh1), 31),
                jnp.where(
                    iota <= 11,
                    lax.bitwise_and(lax.shift_right_logical(lastB, sh2), 31),
                    jnp.where(iota == 12, lastC, 0)))
            cv2 = cv + hist
            mx = cv2
            for d in (8, 4, 2, 1):
                mx = jnp.maximum(
                    mx, mx.at[lax.bitwise_xor(iota, jnp.int32(d))].get(
                        mode="promise_in_bounds"))
            anyf = mx[0] >= FLUSH

            @pl.when(anyf)
            def _():
                for p in range(P):
                    cntp = cv2[p]

                    @pl.when(cntp >= FLUSH)
                    def _():
                        nb = scnt[16 + p]
                        base = (t * P + p) * SEG + nb * FLUSH
                        pltpu.sync_copy(stgf_r.at[pl.ds(p * STG, FLUSH)],
                                        brow_hbm.at[pl.ds(base, FLUSH)])
                        pltpu.sync_copy(stgf_c.at[pl.ds(p * STG, FLUSH)],
                                        bcol_hbm.at[pl.ds(base, FLUSH)])
                        rem = cntp - FLUSH
                        pm = iota < rem
                        tr = stgf_r[pl.ds(p * STG + FLUSH, 16)]
                        tcv = stgf_c[pl.ds(p * STG + FLUSH, 16)]
                        plsc.store_scatter(stgf_r, [p * STG + iota], tr,
                                           mask=pm)
                        plsc.store_scatter(stgf_c, [p * STG + iota], tcv,
                                           mask=pm)
                        scnt[16 + p] = nb + 1

            return jnp.where(cv2 >= FLUSH, cv2 - FLUSH, cv2)

        return lax.fori_loop(0, CH // 16, vec_body, cnt_vec)

    cnt_vec = lax.fori_loop(0, ET // CH, chunk_body,
                            jnp.zeros((16,), jnp.int32))

    cvec = jnp.zeros((16,), jnp.int32)
    for p in range(P):
        cnt = cnt_vec[p]

        def fill_body(j, carry):
            idx16 = j * 16 + iota
            m = idx16 >= cnt
            drow_j = jnp.int32(NNODE) + lax.bitwise_and(idx16 + t * 37,
                                                        jnp.int32(255))
            dcol_j = jnp.int32(RS) + lax.bitwise_and(idx16 + t * 11,
                                                     jnp.int32(127))
            cur_r = stgf_r[pl.ds(p * STG + j * 16, 16)]
            cur_c = stgf_c[pl.ds(p * STG + j * 16, 16)]
            stgf_r[pl.ds(p * STG + j * 16, 16)] = jnp.where(m, drow_j, cur_r)
            stgf_c[pl.ds(p * STG + j * 16, 16)] = jnp.where(m, dcol_j, cur_c)
            return carry

        lax.fori_loop(0, FLUSH // 16, fill_body, 0)
        nb = scnt[16 + p]

        @pl.when(cnt > 0)
        def _():
            base = (t * P + p) * SEG + nb * FLUSH
            pltpu.sync_copy(stgf_r.at[pl.ds(p * STG, FLUSH)],
                            brow_hbm.at[pl.ds(base, FLUSH)])
            pltpu.sync_copy(stgf_c.at[pl.ds(p * STG, FLUSH)],
                            bcol_hbm.at[pl.ds(base, FLUSH)])

        nbf = jnp.where(cnt > 0, nb + 1, nb)
        cvec = jnp.where(iota == p, nbf, cvec)

    cntbuf[...] = cvec
    pltpu.sync_copy(cntbuf, counts_hbm.at[t])

    @pl.when(t == 0)
    def _():
        def fillall(j, carry):
            idx16 = j * 16 + iota
            stgf_r[pl.ds(j * 16, 16)] = jnp.int32(NNODE) + lax.bitwise_and(
                idx16, jnp.int32(255))
            stgf_c[pl.ds(j * 16, 16)] = jnp.int32(RS) + lax.bitwise_and(
                idx16, jnp.int32(127))
            return carry

        lax.fori_loop(0, FLUSH // 16, fillall, 0)
        pltpu.sync_copy(stgf_r.at[pl.ds(0, FLUSH)],
                        brow_hbm.at[pl.ds(NW * P * SEG, FLUSH)])
        pltpu.sync_copy(stgf_c.at[pl.ds(0, FLUSH)],
                        bcol_hbm.at[pl.ds(NW * P * SEG, FLUSH)])


def _bin_edges(rows, cols):
    k = pl.kernel(
        _bin_body,
        out_type=[
            jax.ShapeDtypeStruct((TOTE + FLUSH,), jnp.int32),
            jax.ShapeDtypeStruct((TOTE + FLUSH,), jnp.int32),
            jax.ShapeDtypeStruct((NW, 16), jnp.int32),
        ],
        mesh=_mesh,
        compiler_params=pltpu.CompilerParams(needs_layout_passes=False, use_tc_tiling_on_sc=False),
        scratch_types=[
            pltpu.VMEM((CH,), jnp.int32),
            pltpu.VMEM((CH,), jnp.int32),
            pltpu.VMEM((P * STG,), jnp.int32),
            pltpu.VMEM((P * STG,), jnp.int32),
            pltpu.VMEM((16,), jnp.int32),
            pltpu.SMEM((32,), jnp.int32),
        ],
    )
    return k(rows, cols)


def _deg_body(bcol_hbm, counts_hbm, deg_hbm,
              colb, ones, zbuf, cntf, worklist, deg_sp, sema):
    c = lax.axis_index("c")
    s = lax.axis_index("s")
    iota = lax.iota(jnp.int32, 16)
    pltpu.sync_copy(counts_hbm, cntf)

    def zb(i, carry):
        zbuf[pl.ds(i * 16, 16)] = jnp.zeros((16,), jnp.float32)
        return carry

    lax.fori_loop(0, 640 // 16, zb, 0)

    def ob(i, carry):
        ones[pl.ds(i * 16, 16)] = jnp.ones((16,), jnp.float32)
        return carry

    lax.fori_loop(0, 128 // 16, ob, 0)

    def pp_body(pp, carry_pp):
        pv = pp * NC + c

        @pl.when(pv < P)
        def _():
            @pl.when(s < SPAD // 640)
            def _():
                pltpu.sync_copy(zbuf, deg_sp.at[pl.ds(s * 640, 640)])
            ngrp = _build_worklist(pv, s, cntf, worklist, iota)
            plsc.subcore_barrier()

            def grp_body(g, carry):
                wv = worklist[pl.ds(g * 16, 16)]
                for j in range(16):
                    blk = wv[j]
                    pltpu.sync_copy(bcol_hbm.at[blk], colb)
                    ds_ = [pltpu.async_copy(ones, deg_sp.at[colb.at[jj]],
                                            sema, add=True) for jj in range(4)]
                    for d in ds_:
                        d.wait()
                return carry

            lax.fori_loop(0, ngrp, grp_body, 0)
            plsc.subcore_barrier()
            pltpu.sync_copy(deg_sp.at[pl.ds(s * 512, 512)],
                            deg_hbm.at[pl.ds(pv * RS + s * 512, 512)])
            plsc.subcore_barrier()

        return carry_pp

    lax.fori_loop(0, _pp_count(), pp_body, 0)


def _degrees(bcol_blk, counts_flat):
    k = pl.kernel(
        _deg_body,
        out_type=[jax.ShapeDtypeStruct((NOUT,), jnp.float32)],
        mesh=_mesh,
        compiler_params=pltpu.CompilerParams(needs_layout_passes=False, use_tc_tiling_on_sc=False),
        scratch_types=[
            pltpu.VMEM((4, 128), jnp.int32),
            pltpu.VMEM((128,), jnp.float32),
            pltpu.VMEM((640,), jnp.float32),
            pltpu.VMEM((P * NW,), jnp.int32),
            pltpu.VMEM((WL,), jnp.int32),
            pltpu.VMEM_SHARED((SPAD,), jnp.float32),
            pltpu.SemaphoreType.DMA,
        ],
    )
    (deg,) = k(bcol_blk, counts_flat)
    return deg


ZCH = 104


def _acc_body(F, D, y_hbm, brow_hbm, bcol_hbm, counts_hbm, s_hbm,
              idxr, idxc, msg, zbuf, cntf, worklist, s_sp, semi, semg, sems):
    c = lax.axis_index("c")
    s = lax.axis_index("s")
    iota = lax.iota(jnp.int32, 16)
    pltpu.sync_copy(counts_hbm, cntf)

    def zb(r, carry):
        for cc in range(F // 16):
            zbuf[r, pl.ds(cc * 16, 16)] = jnp.zeros((16,), jnp.float32)
        return carry

    lax.fori_loop(0, ZCH, zb, 0)

    def pp_body(pp, carry_pp):
        pv = pp * NC + c

        @pl.when(pv < P)
        def _():
            for q in range(5):
                pltpu.sync_copy(zbuf, s_sp.at[pl.ds((s * 5 + q) * ZCH, ZCH)])
            ngrp = _build_worklist(pv, s, cntf, worklist, iota)
            plsc.subcore_barrier()

            def grp_body(g, carry):
                wv = worklist[pl.ds(g * 16, 16)]
                di = []
                for j in range(16):
                    blk = wv[j]
                    di.append(pltpu.async_copy(brow_hbm.at[blk], idxr.at[j],
                                               semi))
                    di.append(pltpu.async_copy(bcol_hbm.at[blk], idxc.at[j],
                                               semi))
                for d in di:
                    d.wait()

                gd = [None] * 16
                sd = [None] * 16

                def issue_gather(j):
                    slot = j % D
                    gd[j] = [pltpu.async_copy(
                        y_hbm.at[idxr.at[j, jj]],
                        msg.at[slot, pl.ds(jj * 128, 128)], semg[slot])
                        for jj in range(4)]

                def issue_scatter(j):
                    slot = j % D
                    for d in gd[j]:
                        d.wait()
                    sd[j] = [pltpu.async_copy(
                        msg.at[slot, pl.ds(jj * 128, 128)],
                        s_sp.at[idxc.at[j, jj]], sems[slot], add=True)
                        for jj in range(4)]

                for j in range(16):
                    if j >= D:
                        for d in sd[j - D]:
                            d.wait()
                    issue_gather(j)
                    if j >= D - 1:
                        issue_scatter(j - (D - 1))
                for j in range(17 - D, 16):
                    issue_scatter(j)
                for j in range(16 - D, 16):
                    for d in sd[j]:
                        d.wait()
                return carry

            lax.fori_loop(0, ngrp, grp_body, 0)
            plsc.subcore_barrier()
            pltpu.sync_copy(s_sp.at[pl.ds(s * 512, 512)],
                            s_hbm.at[pl.ds(pv * RS + s * 512, 512)])
            plsc.subcore_barrier()

        return carry_pp

    lax.fori_loop(0, _pp_count(), pp_body, 0)


def _accumulate(y, brow_blk, bcol_blk, counts_flat, F):
    D = 2 if F > 32 else 4
    k = pl.kernel(
        functools.partial(_acc_body, F, D),
        out_type=[jax.ShapeDtypeStruct((NOUT, F), jnp.float32)],
        mesh=_mesh,
        compiler_params=pltpu.CompilerParams(needs_layout_passes=False, use_tc_tiling_on_sc=False),
        scratch_types=[
            pltpu.VMEM((16, 4, 128), jnp.int32),
            pltpu.VMEM((16, 4, 128), jnp.int32),
            pltpu.VMEM((D, 512, F), jnp.float32),
            pltpu.VMEM((ZCH, F), jnp.float32),
            pltpu.VMEM((P * NW,), jnp.int32),
            pltpu.VMEM((WL,), jnp.int32),
            pltpu.VMEM_SHARED((SPAD, F), jnp.float32),
            pltpu.SemaphoreType.DMA,
            [pltpu.SemaphoreType.DMA for _ in range(D)],
            [pltpu.SemaphoreType.DMA for _ in range(D)],
        ],
    )
    (out,) = k(y, brow_blk, bcol_blk, counts_flat)
    return out


BR = 1024
GRID = (NPAD + BR - 1) // BR


def _tc_first(deg2, x, W1):
    def f(deg_ref, x_ref, w_ref, dis_ref, y_ref):
        dis = lax.rsqrt(deg_ref[...] + 1.0)
        dis_ref[...] = dis
        y_ref[...] = jnp.dot(x_ref[...], w_ref[...],
                             preferred_element_type=jnp.float32) * dis

    return pl.pallas_call(
        f,
        grid=(GRID,),
        in_specs=[
            pl.BlockSpec((BR, 1), lambda i: (i, 0)),
            pl.BlockSpec((BR, 21), lambda i: (i, 0)),
            pl.BlockSpec((21, 32), lambda i: (0, 0)),
        ],
        out_specs=[
            pl.BlockSpec((BR, 1), lambda i: (i, 0)),
            pl.BlockSpec((BR, 32), lambda i: (i, 0)),
        ],
        out_shape=[
            jax.ShapeDtypeStruct((NPAD, 1), jnp.float32),
            jax.ShapeDtypeStruct((NPAD, 32), jnp.float32),
        ],
    )(deg2, x, W1)


def _tc_mid(sarr, y, dis, b, W, Fin, Fout):
    def f(s_ref, y_ref, d_ref, b_ref, w_ref, o_ref):
        d = d_ref[...]
        h = jnp.maximum(d * (s_ref[...] + y_ref[...]) + b_ref[...], 0.0)
        o_ref[...] = jnp.dot(h, w_ref[...],
                             preferred_element_type=jnp.float32) * d

    return pl.pallas_call(
        f,
        grid=(GRID,),
        in_specs=[
            pl.BlockSpec((BR, Fin), lambda i: (i, 0)),
            pl.BlockSpec((BR, Fin), lambda i: (i, 0)),
            pl.BlockSpec((BR, 1), lambda i: (i, 0)),
            pl.BlockSpec((1, Fin), lambda i: (0, 0)),
            pl.BlockSpec((Fin, Fout), lambda i: (0, 0)),
        ],
        out_specs=pl.BlockSpec((BR, Fout), lambda i: (i, 0)),
        out_shape=jax.ShapeDtypeStruct((NPAD, Fout), jnp.float32),
    )(sarr, y, dis, b, W)


def _tc_head(s3, y3, dis, b3, Wl1, bl1, Wl2, bl2):
    def f(s_ref, y_ref, d_ref, b_ref, w1_ref, c1_ref, w2_ref, c2_ref, o_ref):
        d = d_ref[...]
        h = jnp.maximum(d * (s_ref[...] + y_ref[...]) + b_ref[...], 0.0)
        z = jnp.maximum(jnp.dot(h, w1_ref[...],
                                preferred_element_type=jnp.float32)
                        + c1_ref[...], 0.0)
        o_ref[...] = jnp.dot(z, w2_ref[...],
                             preferred_element_type=jnp.float32) + c2_ref[...]

    return pl.pallas_call(
        f,
        grid=(GRID,),
        in_specs=[
            pl.BlockSpec((BR, 32), lambda i: (i, 0)),
            pl.BlockSpec((BR, 32), lambda i: (i, 0)),
            pl.BlockSpec((BR, 1), lambda i: (i, 0)),
            pl.BlockSpec((1, 32), lambda i: (0, 0)),
            pl.BlockSpec((32, 20), lambda i: (0, 0)),
            pl.BlockSpec((1, 20), lambda i: (0, 0)),
            pl.BlockSpec((20, 1), lambda i: (0, 0)),
            pl.BlockSpec((1, 1), lambda i: (0, 0)),
        ],
        out_specs=pl.BlockSpec((BR, 1), lambda i: (i, 0)),
        out_shape=jax.ShapeDtypeStruct((NNODE, 1), jnp.float32),
    )(s3, y3, dis, b3, Wl1, bl1, Wl2, bl2)


def kernel(x, edge_index, W1, b1, W2, b2, W3, b3, Wl1, bl1, Wl2, bl2):
    rows = edge_index[0]
    cols = edge_index[1]

    brow, bcol, counts = _bin_edges(rows, cols)
    brow_blk = brow.reshape(NBLKTOT + 1, 4, 128)
    bcol_blk = bcol.reshape(NBLKTOT + 1, 4, 128)
    counts_flat = counts[:, :P].T.reshape(P * NW)

    deg = _degrees(bcol_blk, counts_flat)

    dis, y1 = _tc_first(deg.reshape(NOUT, 1), x, W1)
    s1 = _accumulate(y1, brow_blk, bcol_blk, counts_flat, 32)
    y2 = _tc_mid(s1, y1, dis, b1.reshape(1, -1), W2, 32, 64)
    s2 = _accumulate(y2, brow_blk, bcol_blk, counts_flat, 64)
    y3 = _tc_mid(s2, y2, dis, b2.reshape(1, -1), W3, 64, 32)
    s3 = _accumulate(y3, brow_blk, bcol_blk, counts_flat, 32)
    return _tc_head(s3, y3, dis, b3.reshape(1, -1), Wl1, bl1.reshape(1, -1),
                    Wl2, bl2.reshape(1, -1))


# speedup vs baseline: 3.5931x; 1.0214x over previous
import functools

import jax
import jax.numpy as jnp
from jax import lax
from jax.experimental import pallas as pl
from jax.experimental.pallas import tpu as pltpu
from jax.experimental.pallas import tpu_sc as plsc

NNODE = 100000
NEDGE = 3200000

NC = 2
NS = 16
NW = NC * NS

RS = 8192
P = (NNODE + RS - 1) // RS
SPAD = RS + 128
NPAD = 100352
NOUT = P * RS

ET = NEDGE // NW
CH = 2000
FLUSH = 512
STG = FLUSH + 16
NBLK_SEG = ET // FLUSH + 1
SEG = NBLK_SEG * FLUSH
TOTE = NW * P * SEG
NBLKTOT = NW * P * NBLK_SEG
SZCH = 344
WL = 544
DUMMY_BLK = NBLKTOT

_mesh = plsc.VectorSubcoreMesh(core_axis_name="c", subcore_axis_name="s")


def _prefix16(x, iota):
    y = x
    for d in (1, 2, 4, 8):
        idx = jnp.maximum(iota - d, 0)
        sh = y.at[idx].get(mode="promise_in_bounds")
        y = y + jnp.where(iota >= d, sh, 0)
    return y


def _build_worklist(pv, s, cntf, worklist, iota):
    cr0 = cntf[pl.ds(pv * NW, 16)]
    cr1 = cntf[pl.ds(pv * NW + 16, 16)]
    wcnt = jnp.int32(0)
    gbase = jnp.int32(0)
    for t2 in range(NW):
        v = cr0 if t2 < 16 else cr1
        nb = v[t2 % 16]
        b0 = lax.rem(s - gbase, jnp.int32(16))
        b0 = jnp.where(b0 < 0, b0 + 16, b0)
        nmy = jnp.maximum((nb - b0 + 15) // 16, 0)
        cand = (t2 * P) * NBLK_SEG + pv * NBLK_SEG + b0 + iota * 16
        plsc.store_scatter(worklist, [wcnt + iota], cand, mask=iota < nmy)
        wcnt = wcnt + nmy
        gbase = gbase + nb
    npad = lax.rem(jnp.int32(16) - lax.rem(wcnt, jnp.int32(16)), jnp.int32(16))
    plsc.store_scatter(worklist, [wcnt + iota],
                       jnp.full((16,), DUMMY_BLK, jnp.int32),
                       mask=iota < npad)
    return (wcnt + npad) // 16


def _pp_count():
    return (P + NC - 1) // NC


def _bin_body(rows_hbm, cols_hbm, brow_hbm, bcol_hbm, counts_hbm,
              rowch, colch, stgf_r, stgf_c, cntbuf, scnt):
    c = lax.axis_index("c")
    s = lax.axis_index("s")
    t = c * NS + s
    e0 = t * ET
    iota = lax.iota(jnp.int32, 16)
    ones16 = jnp.full((16,), 1, jnp.int32)

    for p in range(P):
        scnt[16 + p] = jnp.int32(0)

    def chunk_body(k, cnt_vec):
        pltpu.sync_copy(rows_hbm.at[pl.ds(e0 + k * CH, CH)], rowch)
        pltpu.sync_copy(cols_hbm.at[pl.ds(e0 + k * CH, CH)], colch)

        def vec_body(v, cv):
            r16 = rowch[pl.ds(v * 16, 16)]
            c16 = colch[pl.ds(v * 16, 16)]
            p16 = lax.shift_right_logical(c16, 13)
            l16 = lax.bitwise_and(c16, RS - 1)
            isA = p16 <= 5
            isB = (p16 >= 6) & (p16 <= 11)
            shA = jnp.where(isA, 5 * p16, 0)
            shB = jnp.where(isB, 5 * (p16 - 6), 0)
            encA = jnp.where(isA, lax.shift_left(ones16, shA), 0)
            encB = jnp.where(isB, lax.shift_left(ones16, shB), 0)
            encC = jnp.where(p16 == 12, ones16, 0)
            pA = _prefix16(encA, iota)
            pB = _prefix16(encB, iota)
            pC = _prefix16(encC, iota)
            rank = jnp.where(
                isA, lax.bitwise_and(lax.shift_right_logical(pA, shA), 31),
                jnp.where(
                    isB, lax.bitwise_and(lax.shift_right_logical(pB, shB), 31),
                    pC))
            cntg = cv.at[p16].get(mode="promise_in_bounds")
            dest = p16 * STG + cntg + rank - 1
            plsc.store_scatter(stgf_r, [dest], r16)
            plsc.store_scatter(stgf_c, [dest], l16)
            lastA = pA[15]
            lastB = pB[15]
            lastC = pC[15]
            sh1 = 5 * jnp.minimum(iota, 5)
            sh2 = 5 * jnp.clip(iota - 6, 0, 5)
            hist = jnp.where(
                iota <= 5,
                lax.bitwise_and(lax.shift_right_logical(lastA, sh1), 31),
                jnp.where(
                    iota <= 11,
                    lax.bitwise_and(lax.shift_right_logical(lastB, sh2), 31),
                    jnp.where(iota == 12, lastC, 0)))
            cv2 = cv + hist
            mx = cv2
            for d in (8, 4, 2, 1):
                mx = jnp.maximum(
                    mx, mx.at[lax.bitwise_xor(iota, jnp.int32(d))].get(
                        mode="promise_in_bounds"))
            anyf = mx[0] >= FLUSH

            @pl.when(anyf)
            def _():
                for p in range(P):
                    cntp = cv2[p]

                    @pl.when(cntp >= FLUSH)
                    def _():
                        nb = scnt[16 + p]
                        base = (t * P + p) * SEG + nb * FLUSH
                        pltpu.sync_copy(stgf_r.at[pl.ds(p * STG, FLUSH)],
                                        brow_hbm.at[pl.ds(base, FLUSH)])
                        pltpu.sync_copy(stgf_c.at[pl.ds(p * STG, FLUSH)],
                                        bcol_hbm.at[pl.ds(base, FLUSH)])
                        rem = cntp - FLUSH
                        pm = iota < rem
                        tr = stgf_r[pl.ds(p * STG + FLUSH, 16)]
                        tcv = stgf_c[pl.ds(p * STG + FLUSH, 16)]
                        plsc.store_scatter(stgf_r, [p * STG + iota], tr,
                                           mask=pm)
                        plsc.store_scatter(stgf_c, [p * STG + iota], tcv,
                                           mask=pm)
                        scnt[16 + p] = nb + 1

            return jnp.where(cv2 >= FLUSH, cv2 - FLUSH, cv2)

        return lax.fori_loop(0, CH // 16, vec_body, cnt_vec)

    cnt_vec = lax.fori_loop(0, ET // CH, chunk_body,
                            jnp.zeros((16,), jnp.int32))

    cvec = jnp.zeros((16,), jnp.int32)
    for p in range(P):
        cnt = cnt_vec[p]

        def fill_body(j, carry):
            idx16 = j * 16 + iota
            m = idx16 >= cnt
            drow_j = jnp.int32(NNODE) + lax.bitwise_and(idx16 + t * 37,
                                                        jnp.int32(255))
            dcol_j = jnp.int32(RS) + lax.bitwise_and(idx16 + t * 11,
                                                     jnp.int32(127))
            cur_r = stgf_r[pl.ds(p * STG + j * 16, 16)]
            cur_c = stgf_c[pl.ds(p * STG + j * 16, 16)]
            stgf_r[pl.ds(p * STG + j * 16, 16)] = jnp.where(m, drow_j, cur_r)
            stgf_c[pl.ds(p * STG + j * 16, 16)] = jnp.where(m, dcol_j, cur_c)
            return carry

        lax.fori_loop(0, FLUSH // 16, fill_body, 0)
        nb = scnt[16 + p]

        @pl.when(cnt > 0)
        def _():
            base = (t * P + p) * SEG + nb * FLUSH
            pltpu.sync_copy(stgf_r.at[pl.ds(p * STG, FLUSH)],
                            brow_hbm.at[pl.ds(base, FLUSH)])
            pltpu.sync_copy(stgf_c.at[pl.ds(p * STG, FLUSH)],
                            bcol_hbm.at[pl.ds(base, FLUSH)])

        nbf = jnp.where(cnt > 0, nb + 1, nb)
        cvec = jnp.where(iota == p, nbf, cvec)

    cntbuf[...] = cvec
    pltpu.sync_copy(cntbuf, counts_hbm.at[t])

    @pl.when(t == 0)
    def _():
        def fillall(j, carry):
            idx16 = j * 16 + iota
            stgf_r[pl.ds(j * 16, 16)] = jnp.int32(NNODE) + lax.bitwise_and(
                idx16, jnp.int32(255))
            stgf_c[pl.ds(j * 16, 16)] = jnp.int32(RS) + lax.bitwise_and(
                idx16, jnp.int32(127))
            return carry

        lax.fori_loop(0, FLUSH // 16, fillall, 0)
        pltpu.sync_copy(stgf_r.at[pl.ds(0, FLUSH)],
                        brow_hbm.at[pl.ds(NW * P * SEG, FLUSH)])
        pltpu.sync_copy(stgf_c.at[pl.ds(0, FLUSH)],
                        bcol_hbm.at[pl.ds(NW * P * SEG, FLUSH)])


def _bin_edges(rows, cols):
    k = pl.kernel(
        _bin_body,
        out_type=[
            jax.ShapeDtypeStruct((TOTE + FLUSH,), jnp.int32),
            jax.ShapeDtypeStruct((TOTE + FLUSH,), jnp.int32),
            jax.ShapeDtypeStruct((NW, 16), jnp.int32),
        ],
        mesh=_mesh,
        compiler_params=pltpu.CompilerParams(needs_layout_passes=False, use_tc_tiling_on_sc=False),
        scratch_types=[
            pltpu.VMEM((CH,), jnp.int32),
            pltpu.VMEM((CH,), jnp.int32),
            pltpu.VMEM((P * STG,), jnp.int32),
            pltpu.VMEM((P * STG,), jnp.int32),
            pltpu.VMEM((16,), jnp.int32),
            pltpu.SMEM((32,), jnp.int32),
        ],
    )
    return k(rows, cols)


def _deg_body(bcol_hbm, counts_hbm, deg_hbm,
              colb, ones, zbuf, cntf, worklist, deg_sp, semi, sema):
    c = lax.axis_index("c")
    s = lax.axis_index("s")
    iota = lax.iota(jnp.int32, 16)
    pltpu.sync_copy(counts_hbm, cntf)

    def zb(i, carry):
        zbuf[pl.ds(i * 16, 16)] = jnp.zeros((16,), jnp.float32)
        return carry

    lax.fori_loop(0, 640 // 16, zb, 0)

    def ob(i, carry):
        ones[pl.ds(i * 16, 16)] = jnp.ones((16,), jnp.float32)
        return carry

    lax.fori_loop(0, 128 // 16, ob, 0)

    def pp_body(pp, carry_pp):
        pv = pp * NC + c

        @pl.when(pv < P)
        def _():
            @pl.when(s < SPAD // 640)
            def _():
                pltpu.sync_copy(zbuf, deg_sp.at[pl.ds(s * 640, 640)])
            ngrp = _build_worklist(pv, s, cntf, worklist, iota)
            plsc.subcore_barrier()

            def grp_body(g, carry):
                wv = worklist[pl.ds(g * 16, 16)]
                di = [pltpu.async_copy(bcol_hbm.at[wv[j]], colb.at[j], semi)
                      for j in range(16)]
                for d in di:
                    d.wait()
                ds_ = [pltpu.async_copy(ones, deg_sp.at[colb.at[j, jj]],
                                        sema, add=True)
                       for j in range(16) for jj in range(4)]
                for d in ds_:
                    d.wait()
                return carry

            lax.fori_loop(0, ngrp, grp_body, 0)
            plsc.subcore_barrier()
            pltpu.sync_copy(deg_sp.at[pl.ds(s * 512, 512)],
                            deg_hbm.at[pl.ds(pv * RS + s * 512, 512)])
            plsc.subcore_barrier()

        return carry_pp

    lax.fori_loop(0, _pp_count(), pp_body, 0)


def _degrees(bcol_blk, counts_flat):
    k = pl.kernel(
        _deg_body,
        out_type=[jax.ShapeDtypeStruct((NOUT,), jnp.float32)],
        mesh=_mesh,
        compiler_params=pltpu.CompilerParams(needs_layout_passes=False, use_tc_tiling_on_sc=False),
        scratch_types=[
            pltpu.VMEM((16, 4, 128), jnp.int32),
            pltpu.VMEM((128,), jnp.float32),
            pltpu.VMEM((640,), jnp.float32),
            pltpu.VMEM((P * NW,), jnp.int32),
            pltpu.VMEM((WL,), jnp.int32),
            pltpu.VMEM_SHARED((SPAD,), jnp.float32),
            pltpu.SemaphoreType.DMA,
            pltpu.SemaphoreType.DMA,
        ],
    )
    (deg,) = k(bcol_blk, counts_flat)
    return deg


ZCH = 104


def _acc_body(F, D, y_hbm, brow_hbm, bcol_hbm, counts_hbm, s_hbm,
              idxr, idxc, msg, zbuf, cntf, worklist, s_sp, semi, semg, sems):
    c = lax.axis_index("c")
    s = lax.axis_index("s")
    iota = lax.iota(jnp.int32, 16)
    pltpu.sync_copy(counts_hbm, cntf)

    def zb(r, carry):
        for cc in range(F // 16):
            zbuf[r, pl.ds(cc * 16, 16)] = jnp.zeros((16,), jnp.float32)
        return carry

    lax.fori_loop(0, ZCH, zb, 0)

    def pp_body(pp, carry_pp):
        pv = pp * NC + c

        @pl.when(pv < P)
        def _():
            for q in range(5):
                pltpu.sync_copy(zbuf, s_sp.at[pl.ds((s * 5 + q) * ZCH, ZCH)])
            ngrp = _build_worklist(pv, s, cntf, worklist, iota)
            plsc.subcore_barrier()

            def grp_body(g, carry):
                wv = worklist[pl.ds(g * 16, 16)]
                di = []
                for j in range(16):
                    blk = wv[j]
                    di.append(pltpu.async_copy(brow_hbm.at[blk], idxr.at[j],
                                               semi))
                    di.append(pltpu.async_copy(bcol_hbm.at[blk], idxc.at[j],
                                               semi))
                for d in di:
                    d.wait()

                gd = [None] * 16
                sd = [None] * 16

                def issue_gather(j):
                    slot = j % D
                    gd[j] = [pltpu.async_copy(
                        y_hbm.at[idxr.at[j, jj]],
                        msg.at[slot, pl.ds(jj * 128, 128)], semg[slot])
                        for jj in range(4)]

                def issue_scatter(j):
                    slot = j % D
                    for d in gd[j]:
                        d.wait()
                    sd[j] = [pltpu.async_copy(
                        msg.at[slot, pl.ds(jj * 128, 128)],
                        s_sp.at[idxc.at[j, jj]], sems[slot], add=True)
                        for jj in range(4)]

                for j in range(16):
                    if j >= D:
                        for d in sd[j - D]:
                            d.wait()
                    issue_gather(j)
                    if j >= D - 1:
                        issue_scatter(j - (D - 1))
                for j in range(17 - D, 16):
                    issue_scatter(j)
                for j in range(16 - D, 16):
                    for d in sd[j]:
                        d.wait()
                return carry

            lax.fori_loop(0, ngrp, grp_body, 0)
            plsc.subcore_barrier()
            pltpu.sync_copy(s_sp.at[pl.ds(s * 512, 512)],
                            s_hbm.at[pl.ds(pv * RS + s * 512, 512)])
            plsc.subcore_barrier()

        return carry_pp

    lax.fori_loop(0, _pp_count(), pp_body, 0)


def _accumulate(y, brow_blk, bcol_blk, counts_flat, F):
    D = 2 if F > 32 else 4
    k = pl.kernel(
        functools.partial(_acc_body, F, D),
        out_type=[jax.ShapeDtypeStruct((NOUT, F), jnp.float32)],
        mesh=_mesh,
        compiler_params=pltpu.CompilerParams(needs_layout_passes=False, use_tc_tiling_on_sc=False),
        scratch_types=[
            pltpu.VMEM((16, 4, 128), jnp.int32),
            pltpu.VMEM((16, 4, 128), jnp.int32),
            pltpu.VMEM((D, 512, F), jnp.float32),
            pltpu.VMEM((ZCH, F), jnp.float32),
            pltpu.VMEM((P * NW,), jnp.int32),
            pltpu.VMEM((WL,), jnp.int32),
            pltpu.VMEM_SHARED((SPAD, F), jnp.float32),
            pltpu.SemaphoreType.DMA,
            [pltpu.SemaphoreType.DMA for _ in range(D)],
            [pltpu.SemaphoreType.DMA for _ in range(D)],
        ],
    )
    (out,) = k(y, brow_blk, bcol_blk, counts_flat)
    return out


BR = 1024
GRID = (NPAD + BR - 1) // BR


def _tc_first(deg2, x, W1):
    def f(deg_ref, x_ref, w_ref, dis_ref, y_ref):
        dis = lax.rsqrt(deg_ref[...] + 1.0)
        dis_ref[...] = dis
        y_ref[...] = jnp.dot(x_ref[...], w_ref[...],
                             preferred_element_type=jnp.float32) * dis

    return pl.pallas_call(
        f,
        grid=(GRID,),
        in_specs=[
            pl.BlockSpec((BR, 1), lambda i: (i, 0)),
            pl.BlockSpec((BR, 21), lambda i: (i, 0)),
            pl.BlockSpec((21, 32), lambda i: (0, 0)),
        ],
        out_specs=[
            pl.BlockSpec((BR, 1), lambda i: (i, 0)),
            pl.BlockSpec((BR, 32), lambda i: (i, 0)),
        ],
        out_shape=[
            jax.ShapeDtypeStruct((NPAD, 1), jnp.float32),
            jax.ShapeDtypeStruct((NPAD, 32), jnp.float32),
        ],
    )(deg2, x, W1)


def _tc_mid(sarr, y, dis, b, W, Fin, Fout):
    def f(s_ref, y_ref, d_ref, b_ref, w_ref, o_ref):
        d = d_ref[...]
        h = jnp.maximum(d * (s_ref[...] + y_ref[...]) + b_ref[...], 0.0)
        o_ref[...] = jnp.dot(h, w_ref[...],
                             preferred_element_type=jnp.float32) * d

    return pl.pallas_call(
        f,
        grid=(GRID,),
        in_specs=[
            pl.BlockSpec((BR, Fin), lambda i: (i, 0)),
            pl.BlockSpec((BR, Fin), lambda i: (i, 0)),
            pl.BlockSpec((BR, 1), lambda i: (i, 0)),
            pl.BlockSpec((1, Fin), lambda i: (0, 0)),
            pl.BlockSpec((Fin, Fout), lambda i: (0, 0)),
        ],
        out_specs=pl.BlockSpec((BR, Fout), lambda i: (i, 0)),
        out_shape=jax.ShapeDtypeStruct((NPAD, Fout), jnp.float32),
    )(sarr, y, dis, b, W)


def _tc_head(s3, y3, dis, b3, Wl1, bl1, Wl2, bl2):
    def f(s_ref, y_ref, d_ref, b_ref, w1_ref, c1_ref, w2_ref, c2_ref, o_ref):
        d = d_ref[...]
        h = jnp.maximum(d * (s_ref[...] + y_ref[...]) + b_ref[...], 0.0)
        z = jnp.maximum(jnp.dot(h, w1_ref[...],
                                preferred_element_type=jnp.float32)
                        + c1_ref[...], 0.0)
        o_ref[...] = jnp.dot(z, w2_ref[...],
                             preferred_element_type=jnp.float32) + c2_ref[...]

    return pl.pallas_call(
        f,
        grid=(GRID,),
        in_specs=[
            pl.BlockSpec((BR, 32), lambda i: (i, 0)),
            pl.BlockSpec((BR, 32), lambda i: (i, 0)),
            pl.BlockSpec((BR, 1), lambda i: (i, 0)),
            pl.BlockSpec((1, 32), lambda i: (0, 0)),
            pl.BlockSpec((32, 20), lambda i: (0, 0)),
            pl.BlockSpec((1, 20), lambda i: (0, 0)),
            pl.BlockSpec((20, 1), lambda i: (0, 0)),
            pl.BlockSpec((1, 1), lambda i: (0, 0)),
        ],
        out_specs=pl.BlockSpec((BR, 1), lambda i: (i, 0)),
        out_shape=jax.ShapeDtypeStruct((NNODE, 1), jnp.float32),
    )(s3, y3, dis, b3, Wl1, bl1, Wl2, bl2)


def kernel(x, edge_index, W1, b1, W2, b2, W3, b3, Wl1, bl1, Wl2, bl2):
    rows = edge_index[0]
    cols = edge_index[1]

    brow, bcol, counts = _bin_edges(rows, cols)
    brow_blk = brow.reshape(NBLKTOT + 1, 4, 128)
    bcol_blk = bcol.reshape(NBLKTOT + 1, 4, 128)
    counts_flat = counts[:, :P].T.reshape(P * NW)

    deg = _degrees(bcol_blk, counts_flat)

    dis, y1 = _tc_first(deg.reshape(NOUT, 1), x, W1)
    s1 = _accumulate(y1, brow_blk, bcol_blk, counts_flat, 32)
    y2 = _tc_mid(s1, y1, dis, b1.reshape(1, -1), W2, 32, 64)
    s2 = _accumulate(y2, brow_blk, bcol_blk, counts_flat, 64)
    y3 = _tc_mid(s2, y2, dis, b2.reshape(1, -1), W3, 64, 32)
    s3 = _accumulate(y3, brow_blk, bcol_blk, counts_flat, 32)
    return _tc_head(s3, y3, dis, b3.reshape(1, -1), Wl1, bl1.reshape(1, -1),
                    Wl2, bl2.reshape(1, -1))
